# baseline (speedup 1.0000x reference)
import functools

import jax
import jax.numpy as jnp
from jax import lax
from jax.experimental import pallas as pl
from jax.experimental.pallas import tpu as pltpu
from jax.experimental.pallas import tpu_sc as plsc

N = 10000
E = 320000
D = 128
DE = 16

NC = 2
NS = 16
NW = NC * NS
L = 16

NPAD = 10240
EB = 128
EW = 10240
BPW = EW // EB
EPAD = EW * NW
EROWS = EPAD // EB
DUMP = NPAD - 1
SHIFT = 14

_mesh = plsc.VectorSubcoreMesh(
    core_axis_name="c", subcore_axis_name="s", num_cores=NC, num_subcores=NS)
_sc_params = pltpu.CompilerParams(needs_layout_passes=False)


def _unpack(pk_v, j, srcb, dstb):
    for k in range(EB // L):
        sl = pl.ds(k * L, L)
        p16 = pk_v[j, sl]
        srcb[sl] = jnp.bitwise_and(p16, (1 << SHIFT) - 1)
        dstb[sl] = lax.shift_right_logical(p16, SHIFT)


def _sc_seg_body(x_hbm, pk_hbm, z2d_hbm,
                 out_hbm,
                 pk_v, rows_a, rows_b, srcb_a, srcb_b, dstb_a, dstb_b, acc_sh,
                 sem_ga, sem_gb, sem_sa, sem_sb):
    c = lax.axis_index("c")
    s = lax.axis_index("s")
    w = c * NS + s
    pltpu.sync_copy(pk_hbm.at[pl.ds(w * BPW, BPW)], pk_v)
    pltpu.sync_copy(z2d_hbm, rows_a)
    nblk = NPAD // EB // NS
    for b in range(nblk):
        pltpu.sync_copy(rows_a, acc_sh.at[pl.ds((s * nblk + b) * EB, EB)])
    _unpack(pk_v, 0, srcb_a, dstb_a)
    pltpu.async_copy(x_hbm.at[srcb_a], rows_a, sem_ga)
    plsc.subcore_barrier()

    npair = BPW // 2

    def pair(g, carry):
        j = 2 * g
        pltpu.make_async_copy(x_hbm.at[srcb_a], rows_a, sem_ga).wait()

        @pl.when(g > 0)
        def _():
            pltpu.make_async_copy(rows_b, acc_sh.at[dstb_b], sem_sb).wait()
        _unpack(pk_v, j + 1, srcb_b, dstb_b)
        pltpu.async_copy(x_hbm.at[srcb_b], rows_b, sem_gb)
        pltpu.async_copy(rows_a, acc_sh.at[dstb_a], sem_sa, add=True)
        pltpu.make_async_copy(x_hbm.at[srcb_b], rows_b, sem_gb).wait()
        pltpu.make_async_copy(rows_a, acc_sh.at[dstb_a], sem_sa).wait()

        @pl.when(g + 1 < npair)
        def _():
            _unpack(pk_v, j + 2, srcb_a, dstb_a)
            pltpu.async_copy(x_hbm.at[srcb_a], rows_a, sem_ga)
        pltpu.async_copy(rows_b, acc_sh.at[dstb_b], sem_sb, add=True)
        return carry

    lax.fori_loop(0, npair, pair, 0)
    pltpu.make_async_copy(rows_b, acc_sh.at[dstb_b], sem_sb).wait()
    plsc.subcore_barrier()
    r0 = s * (NPAD // NS)
    pltpu.sync_copy(acc_sh.at[pl.ds(r0, NPAD // NS)],
                    out_hbm.at[c, pl.ds(r0, NPAD // NS)])


_sc_seg = pl.kernel(
    _sc_seg_body,
    compiler_params=_sc_params,
    out_type=jax.ShapeDtypeStruct((NC, NPAD, D), jnp.float32),
    mesh=_mesh,
    scratch_types=[
        pltpu.VMEM((BPW, EB), jnp.int32),
        pltpu.VMEM((EB, D), jnp.float32),
        pltpu.VMEM((EB, D), jnp.float32),
        pltpu.VMEM((EB,), jnp.int32),
        pltpu.VMEM((EB,), jnp.int32),
        pltpu.VMEM((EB,), jnp.int32),
        pltpu.VMEM((EB,), jnp.int32),
        pltpu.VMEM_SHARED((NPAD, D), jnp.float32),
        pltpu.SemaphoreType.DMA,
        pltpu.SemaphoreType.DMA,
        pltpu.SemaphoreType.DMA,
        pltpu.SemaphoreType.DMA,
    ],
)


def _sc_hist_body(d1_hbm, d2_hbm, d3_hbm, z1d_hbm,
                  hist_hbm,
                  dst_v, hist_v):
    c = lax.axis_index("c")
    s = lax.axis_index("s")
    w = c * NS + s
    ones = jnp.full((L,), 1.0, jnp.float32)
    for r, d_hbm in enumerate((d1_hbm, d2_hbm, d3_hbm)):
        pltpu.sync_copy(d_hbm.at[pl.ds(w * BPW, BPW)], dst_v)
        pltpu.sync_copy(z1d_hbm, hist_v)

        def body(j, carry):
            for k in range(EB // L):
                d16 = dst_v[j, pl.ds(k * L, L)]
                plsc.addupdate_scatter(hist_v, [d16], ones)
            return carry

        lax.fori_loop(0, BPW, body, 0)
        pltpu.sync_copy(hist_v, hist_hbm.at[r, w])


_sc_hist = pl.kernel(
    _sc_hist_body,
    compiler_params=_sc_params,
    out_type=jax.ShapeDtypeStruct((3, NW, NPAD), jnp.float32),
    mesh=_mesh,
    scratch_types=[
        pltpu.VMEM((BPW, EB), jnp.int32),
        pltpu.VMEM((NPAD,), jnp.float32),
    ],
)


def _sc_gat_logits_body(srcv_hbm, dstv_hbm, eav_hbm, hsa_hbm, hda_hbm,
                        shift_hbm, z1d_hbm,
                        ex_hbm, hist_hbm,
                        src_v, dst_v, ea_v, ex_v, hsa_v, hda_v, hist_v,
                        shift_v):
    c = lax.axis_index("c")
    s = lax.axis_index("s")
    w = c * NS + s
    pltpu.sync_copy(srcv_hbm.at[pl.ds(w * BPW, BPW)], src_v)
    pltpu.sync_copy(dstv_hbm.at[pl.ds(w * BPW, BPW)], dst_v)
    pltpu.sync_copy(eav_hbm.at[pl.ds(w * BPW, BPW)], ea_v)
    pltpu.sync_copy(hsa_hbm, hsa_v)
    pltpu.sync_copy(hda_hbm, hda_v)
    pltpu.sync_copy(z1d_hbm, hist_v)
    pltpu.sync_copy(shift_hbm, shift_v)

    def body(j, carry):
        for k in range(EB // L):
            sl = pl.ds(k * L, L)
            s16 = src_v[j, sl]
            d16 = dst_v[j, sl]
            a = (plsc.load_gather(hsa_v, [s16])
                 + plsc.load_gather(hda_v, [d16])
                 + ea_v[j, sl])
            a = jnp.maximum(a, 0.2 * a)
            ex = jnp.exp(a - shift_v[...])
            ex_v[j, sl] = ex
            plsc.addupdate_scatter(hist_v, [d16], ex)
        return carry

    lax.fori_loop(0, BPW, body, 0)
    pltpu.sync_copy(ex_v, ex_hbm.at[pl.ds(w * BPW, BPW)])
    pltpu.sync_copy(hist_v, hist_hbm.at[w])


_sc_gat_logits = pl.kernel(
    _sc_gat_logits_body,
    compiler_params=_sc_params,
    out_type=(
        jax.ShapeDtypeStruct((EROWS, EB), jnp.float32),
        jax.ShapeDtypeStruct((NW, NPAD), jnp.float32),
    ),
    mesh=_mesh,
    scratch_types=[
        pltpu.VMEM((BPW, EB), jnp.int32),
        pltpu.VMEM((BPW, EB), jnp.int32),
        pltpu.VMEM((BPW, EB), jnp.float32),
        pltpu.VMEM((BPW, EB), jnp.float32),
        pltpu.VMEM((NPAD,), jnp.float32),
        pltpu.VMEM((NPAD,), jnp.float32),
        pltpu.VMEM((NPAD,), jnp.float32),
        pltpu.VMEM((L,), jnp.float32),
    ],
)


def _sc_gat_agg_body(hs_hbm, pk_hbm, exv_hbm, deninv_hbm, z2d_hbm,
                     out_hbm,
                     pk_v, rows_v, srcb, dstb, ex_a, ex_b, dinv_v, alpha_v,
                     acc_sh, sem_ea, sem_eb, sem_s):
    c = lax.axis_index("c")
    s = lax.axis_index("s")
    w = c * NS + s
    pltpu.sync_copy(pk_hbm.at[pl.ds(w * BPW, BPW)], pk_v)
    pltpu.sync_copy(deninv_hbm, dinv_v)
    pltpu.sync_copy(z2d_hbm, rows_v)
    nblk = NPAD // EB // NS
    for b in range(nblk):
        pltpu.sync_copy(rows_v, acc_sh.at[pl.ds((s * nblk + b) * EB, EB)])
    pltpu.async_copy(exv_hbm.at[pl.ds(w * BPW, 1)], ex_a, sem_ea)
    plsc.subcore_barrier()

    def half(j, g, ex_cur, sem_cur, ex_nxt, sem_nxt):
        @pl.when(g > 0)
        def _():
            pltpu.make_async_copy(rows_v, acc_sh.at[dstb], sem_s).wait()
        _unpack(pk_v, j, srcb, dstb)
        pltpu.sync_copy(hs_hbm.at[srcb], rows_v)

        @pl.when(j + 1 < BPW)
        def _():
            pltpu.async_copy(exv_hbm.at[pl.ds(w * BPW + j + 1, 1)], ex_nxt,
                             sem_nxt)
        pltpu.make_async_copy(exv_hbm.at[pl.ds(w * BPW + j, 1)], ex_cur,
                              sem_cur).wait()
        for k in range(EB // L):
            sl = pl.ds(k * L, L)
            d16 = dstb[sl]
            alpha_v[sl] = ex_cur[0, sl] * plsc.load_gather(dinv_v, [d16])
        for r in range(EB):
            ar = plsc.load_gather(alpha_v, [jnp.full((L,), r, jnp.int32)])
            for m in range(D // L):
                sl = pl.ds(m * L, L)
                rows_v[r, sl] = rows_v[r, sl] * ar
        pltpu.async_copy(rows_v, acc_sh.at[dstb], sem_s, add=True)

    def pair(g, carry):
        j = 2 * g
        half(j, 2 * g, ex_a, sem_ea, ex_b, sem_eb)
        half(j + 1, 2 * g + 1, ex_b, sem_eb, ex_a, sem_ea)
        return carry

    lax.fori_loop(0, BPW // 2, pair, 0)
    pltpu.make_async_copy(rows_v, acc_sh.at[dstb], sem_s).wait()
    plsc.subcore_barrier()
    r0 = s * (NPAD // NS)
    pltpu.sync_copy(acc_sh.at[pl.ds(r0, NPAD // NS)],
                    out_hbm.at[c, pl.ds(r0, NPAD // NS)])


_sc_gat_agg = pl.kernel(
    _sc_gat_agg_body,
    compiler_params=_sc_params,
    out_type=jax.ShapeDtypeStruct((NC, NPAD, D), jnp.float32),
    mesh=_mesh,
    scratch_types=[
        pltpu.VMEM((BPW, EB), jnp.int32),
        pltpu.VMEM((EB, D), jnp.float32),
        pltpu.VMEM((EB,), jnp.int32),
        pltpu.VMEM((EB,), jnp.int32),
        pltpu.VMEM((1, EB), jnp.float32),
        pltpu.VMEM((1, EB), jnp.float32),
        pltpu.VMEM((NPAD,), jnp.float32),
        pltpu.VMEM((EB,), jnp.float32),
        pltpu.VMEM_SHARED((NPAD, D), jnp.float32),
        pltpu.SemaphoreType.DMA,
        pltpu.SemaphoreType.DMA,
        pltpu.SemaphoreType.DMA,
    ],
)


BT = 512
NG = NPAD // BT


def _tc_dense_body(p_ref, hist_ref, xd_ref, wl_ref, bl_ref, wr_ref, o_ref):
    deg = jnp.sum(hist_ref[...], axis=0)
    dinv = 1.0 / jnp.maximum(deg, 1.0)
    agg = (p_ref[0] + p_ref[1]) * dinv[:, None]
    o_ref[...] = jax.nn.relu(
        jnp.dot(agg, wl_ref[...], preferred_element_type=jnp.float32)
        + bl_ref[...]
        + jnp.dot(xd_ref[...], wr_ref[...], preferred_element_type=jnp.float32))


def _tc_dense(p, hist, xd, wl, bl, wr):
    return pl.pallas_call(
        _tc_dense_body,
        grid=(NG,),
        in_specs=[
            pl.BlockSpec((NC, BT, D), lambda i: (0, i, 0)),
            pl.BlockSpec((NW, BT), lambda i: (0, i)),
            pl.BlockSpec((BT, D), lambda i: (i, 0)),
            pl.BlockSpec((D, D), lambda i: (0, 0)),
            pl.BlockSpec((1, D), lambda i: (0, 0)),
            pl.BlockSpec((D, D), lambda i: (0, 0)),
        ],
        out_specs=pl.BlockSpec((BT, D), lambda i: (i, 0)),
        out_shape=jax.ShapeDtypeStruct((NPAD, D), jnp.float32),
    )(p, hist, xd, wl, bl.reshape(1, D), wr)


def _tc_final_body(p_ref, hist_ref, ph_ref, bg_ref, wl_ref, bl_ref, wr_ref,
                   wm_ref, bm_ref, o_ref):
    deg = jnp.sum(hist_ref[...], axis=0)
    dinv = 1.0 / jnp.maximum(deg, 1.0)
    agg = (p_ref[0] + p_ref[1]) * dinv[:, None]
    h = jax.nn.relu(ph_ref[0] + ph_ref[1] + bg_ref[...])
    in_x = jax.nn.relu(
        jnp.dot(agg, wl_ref[...], preferred_element_type=jnp.float32)
        + bl_ref[...]
        + jnp.dot(h, wr_ref[...], preferred_element_type=jnp.float32))
    o_ref[...] = (jnp.dot(in_x, wm_ref[...], preferred_element_type=jnp.float32)
                  + bm_ref[...])


def _tc_final(p, hist, ph, bg, wl, bl, wr, wm, bm):
    return pl.pallas_call(
        _tc_final_body,
        grid=(NG,),
        in_specs=[
            pl.BlockSpec((NC, BT, D), lambda i: (0, i, 0)),
            pl.BlockSpec((NW, BT), lambda i: (0, i)),
            pl.BlockSpec((NC, BT, D), lambda i: (0, i, 0)),
            pl.BlockSpec((1, D), lambda i: (0, 0)),
            pl.BlockSpec((D, D), lambda i: (0, 0)),
            pl.BlockSpec((1, D), lambda i: (0, 0)),
            pl.BlockSpec((D, D), lambda i: (0, 0)),
            pl.BlockSpec((D, 1), lambda i: (0, 0)),
            pl.BlockSpec((1, 1), lambda i: (0, 0)),
        ],
        out_specs=pl.BlockSpec((BT, 1), lambda i: (i, 0)),
        out_shape=jax.ShapeDtypeStruct((NPAD, 1), jnp.float32),
    )(p, hist, ph, bg.reshape(1, D), wl, bl.reshape(1, D), wr,
      wm, bm.reshape(1, 1))


def _tc_attn_pre_body(g_ref, st_ref, ws_ref, wd_ref, as_ref, ad_ref,
                      hs_ref, hsa_ref, hda_ref, m_ref):
    i = pl.program_id(0)
    hs = jnp.dot(g_ref[...], ws_ref[...], preferred_element_type=jnp.float32)
    hs_ref[...] = hs
    hsa = jnp.dot(hs, as_ref[...], preferred_element_type=jnp.float32)
    wdv = jnp.dot(wd_ref[...], ad_ref[...], preferred_element_type=jnp.float32)
    hda = jnp.dot(st_ref[...], wdv, preferred_element_type=jnp.float32)
    hsa_ref[...] = hsa
    hda_ref[...] = hda

    @pl.when(i == 0)
    def _():
        m_ref[...] = jnp.full((1, 2), -1e30, jnp.float32)

    cur = jnp.concatenate(
        [jnp.max(hsa).reshape(1, 1), jnp.max(hda).reshape(1, 1)], axis=1)
    m_ref[...] = jnp.maximum(m_ref[...], cur)


def _tc_attn_pre(g, st, ws, wd, att_s, att_d):
    return pl.pallas_call(
        _tc_attn_pre_body,
        grid=(NG,),
        in_specs=[
            pl.BlockSpec((BT, D), lambda i: (i, 0)),
            pl.BlockSpec((BT, D), lambda i: (i, 0)),
            pl.BlockSpec((D, D), lambda i: (0, 0)),
            pl.BlockSpec((D, D), lambda i: (0, 0)),
            pl.BlockSpec((D, 1), lambda i: (0, 0)),
            pl.BlockSpec((D, 1), lambda i: (0, 0)),
        ],
        out_specs=(
            pl.BlockSpec((BT, D), lambda i: (i, 0)),
            pl.BlockSpec((BT, 1), lambda i: (i, 0)),
            pl.BlockSpec((BT, 1), lambda i: (i, 0)),
            pl.BlockSpec((1, 2), lambda i: (0, 0)),
        ),
        out_shape=(
            jax.ShapeDtypeStruct((NPAD, D), jnp.float32),
            jax.ShapeDtypeStruct((NPAD, 1), jnp.float32),
            jax.ShapeDtypeStruct((NPAD, 1), jnp.float32),
            jax.ShapeDtypeStruct((1, 2), jnp.float32),
        ),
    )(g, st, ws, wd, att_s.reshape(D, 1), att_d.reshape(D, 1))


BTE = 2000


def _tc_ea_body(ea_ref, we_ref, o_ref, m_ref):
    i = pl.program_id(0)
    v = jnp.sum(ea_ref[...] * we_ref[...], axis=1, keepdims=True)
    o_ref[...] = v

    @pl.when(i == 0)
    def _():
        m_ref[...] = jnp.full((1, 1), -1e30, jnp.float32)

    m_ref[...] = jnp.maximum(m_ref[...], jnp.max(v).reshape(1, 1))


def _tc_ea(edge_attr, wg_e, att_e):
    we = jnp.dot(wg_e, att_e.reshape(D, 1),
                 preferred_element_type=jnp.float32)
    return pl.pallas_call(
        _tc_ea_body,
        grid=(E // BTE,),
        in_specs=[
            pl.BlockSpec((BTE, DE), lambda i: (i, 0)),
            pl.BlockSpec((1, DE), lambda i: (0, 0)),
        ],
        out_specs=(
            pl.BlockSpec((BTE, 1), lambda i: (i, 0)),
            pl.BlockSpec((1, 1), lambda i: (0, 0)),
        ),
        out_shape=(
            jax.ShapeDtypeStruct((E, 1), jnp.float32),
            jax.ShapeDtypeStruct((1, 1), jnp.float32),
        ),
    )(edge_attr, we.reshape(1, DE))


def _tc_deninv_body(hist_ref, o_ref):
    den = jnp.sum(hist_ref[...], axis=0, keepdims=True)
    o_ref[...] = 1.0 / jnp.maximum(den, 1e-16)


def _tc_deninv(hist):
    return pl.pallas_call(
        _tc_deninv_body,
        out_shape=jax.ShapeDtypeStruct((1, NPAD), jnp.float32),
    )(hist)


def _pad_ei(ei):
    src = jnp.concatenate(
        [ei[0], jnp.zeros((EPAD - E,), ei.dtype)]).astype(jnp.int32)
    dst = jnp.concatenate(
        [ei[1], jnp.full((EPAD - E,), DUMP, ei.dtype)]).astype(jnp.int32)
    pk = jnp.bitwise_or(src, jnp.left_shift(dst, SHIFT))
    return (src.reshape(EROWS, EB), dst.reshape(EROWS, EB),
            pk.reshape(EROWS, EB))


def _pad_x(x):
    return jnp.concatenate([x, jnp.zeros((NPAD - N, D), x.dtype)], axis=0)


def kernel(x_game, x_state, edge_attr, Wl, bl, Wr, Wg_s, Wg_d, Wg_e,
           att_s, att_d, att_e, bg, Wm, bm, ei_gg, ei_ss, ei_hist, ei_in):
    z2d = jnp.zeros((EB, D), jnp.float32)
    z1d = jnp.zeros((NPAD,), jnp.float32)
    _, dgg, pgg = _pad_ei(ei_gg)
    _, dss, pss = _pad_ei(ei_ss)
    shh, dhh, phh = _pad_ei(ei_hist)
    _, din, pin_ = _pad_ei(ei_in)
    xg = _pad_x(x_game)
    xs = _pad_x(x_state)

    hist3 = _sc_hist(dgg, dss, din, z1d)
    hist_gg, hist_ss, hist_in = hist3[0], hist3[1], hist3[2]

    p = _sc_seg(xg, pgg, z2d)
    g = _tc_dense(p, hist_gg, xg, Wl[0], bl[0], Wr[0])
    p = _sc_seg(g, pgg, z2d)
    g = _tc_dense(p, hist_gg, g, Wl[1], bl[1], Wr[1])

    p = _sc_seg(xs, pss, z2d)
    st = _tc_dense(p, hist_ss, xs, Wl[2], bl[2], Wr[2])
    p = _sc_seg(st, pss, z2d)
    st = _tc_dense(p, hist_ss, st, Wl[3], bl[3], Wr[3])

    hs, hsa, hda, m12 = _tc_attn_pre(g, st, Wg_s, Wg_d, att_s, att_d)
    ea, m3 = _tc_ea(edge_attr, Wg_e, att_e)
    shift = jnp.maximum(m12[0, 0] + m12[0, 1] + m3[0, 0], 0.0)
    shift16 = jnp.broadcast_to(shift, (L,))
    eav = jnp.concatenate(
        [ea.reshape(E), jnp.zeros((EPAD - E,), jnp.float32)]).reshape(EROWS, EB)
    ex, hist_den = _sc_gat_logits(shh, dhh, eav, hsa.reshape(NPAD),
                                  hda.reshape(NPAD), shift16, z1d)
    deninv = _tc_deninv(hist_den).reshape(NPAD)
    ph = _sc_gat_agg(hs, phh, ex, deninv, z2d)

    p = _sc_seg(g, pin_, z2d)
    out = _tc_final(p, hist_in, ph, bg, Wl[4], bl[4], Wr[4], Wm, bm)
    return out[:N]


# speedup vs baseline: 1.1118x; 1.1118x over previous
import functools

import jax
import jax.numpy as jnp
from jax import lax
from jax.experimental import pallas as pl
from jax.experimental.pallas import tpu as pltpu
from jax.experimental.pallas import tpu_sc as plsc

N = 10000
E = 320000
D = 128
DE = 16

NC = 2
NS = 16
NW = NC * NS
L = 16

NPAD = 10240
EB = 128
EW = 10240
BPW = EW // EB
EPAD = EW * NW
EROWS = EPAD // EB
DUMP = NPAD - 1
SHIFT = 14

_mesh = plsc.VectorSubcoreMesh(
    core_axis_name="c", subcore_axis_name="s", num_cores=NC, num_subcores=NS)
_sc_params = pltpu.CompilerParams(needs_layout_passes=False)


def _unpack(pk_v, j, srcb, dstb):
    for k in range(EB // L):
        sl = pl.ds(k * L, L)
        p16 = pk_v[j, sl]
        srcb[sl] = jnp.bitwise_and(p16, (1 << SHIFT) - 1)
        dstb[sl] = lax.shift_right_logical(p16, SHIFT)


EBS = 64
NBS = EW // EBS


def _sc_seg_body(x_hbm, pk_hbm, z2d_hbm,
                 out_hbm,
                 pk_v, r0, r1, r2, r3, s0, s1, s2, s3, d0, d1, d2, d3, acc_sh,
                 g0, g1, g2, g3, t0, t1, t2, t3):
    c = lax.axis_index("c")
    s = lax.axis_index("s")
    w = c * NS + s
    rows = (r0, r1, r2, r3)
    sb = (s0, s1, s2, s3)
    db = (d0, d1, d2, d3)
    gsem = (g0, g1, g2, g3)
    ssem = (t0, t1, t2, t3)
    pltpu.sync_copy(pk_hbm.at[pl.ds(w * BPW, BPW)], pk_v)
    pltpu.sync_copy(z2d_hbm, r0)
    nblk = NPAD // EBS // NS
    for b in range(nblk):
        pltpu.sync_copy(r0, acc_sh.at[pl.ds((s * nblk + b) * EBS, EBS)])

    def unpack64(j, srcb, dstb):
        row = lax.shift_right_logical(j, 1)
        base = jnp.bitwise_and(j, 1) * EBS
        for k in range(EBS // L):
            sl = pl.ds(base + k * L, L)
            p16 = pk_v[row, sl]
            srcb[pl.ds(k * L, L)] = jnp.bitwise_and(p16, (1 << SHIFT) - 1)
            dstb[pl.ds(k * L, L)] = lax.shift_right_logical(p16, SHIFT)

    for q in range(3):
        unpack64(jnp.int32(q), sb[q], db[q])
        pltpu.async_copy(x_hbm.at[sb[q]], rows[q], gsem[q])
    plsc.subcore_barrier()

    def quad(g, carry):
        for q in range(4):
            j = 4 * g + q
            snx = (q + 3) % 4
            pltpu.make_async_copy(x_hbm.at[sb[q]], rows[q], gsem[q]).wait()
            pltpu.async_copy(rows[q], acc_sh.at[db[q]], ssem[q], add=True)

            @pl.when(j > 0)
            def _():
                pltpu.make_async_copy(rows[snx], acc_sh.at[db[snx]],
                                      ssem[snx]).wait()

            @pl.when(j + 3 < NBS)
            def _():
                unpack64(j + 3, sb[snx], db[snx])
                pltpu.async_copy(x_hbm.at[sb[snx]], rows[snx], gsem[snx])
        return carry

    lax.fori_loop(0, NBS // 4, quad, 0)
    pltpu.make_async_copy(rows[3], acc_sh.at[db[3]], ssem[3]).wait()
    plsc.subcore_barrier()
    rr = s * (NPAD // NS)
    pltpu.sync_copy(acc_sh.at[pl.ds(rr, NPAD // NS)],
                    out_hbm.at[c, pl.ds(rr, NPAD // NS)])


_sc_seg = pl.kernel(
    _sc_seg_body,
    compiler_params=_sc_params,
    out_type=jax.ShapeDtypeStruct((NC, NPAD, D), jnp.float32),
    mesh=_mesh,
    scratch_types=(
        [pltpu.VMEM((BPW, EB), jnp.int32)]
        + [pltpu.VMEM((EBS, D), jnp.float32)] * 4
        + [pltpu.VMEM((EBS,), jnp.int32)] * 8
        + [pltpu.VMEM_SHARED((NPAD, D), jnp.float32)]
        + [pltpu.SemaphoreType.DMA] * 8
    ),
)


def _sc_hist_body(d1_hbm, d2_hbm, d3_hbm, z1d_hbm,
                  hist_hbm,
                  dst_v, hist_v):
    c = lax.axis_index("c")
    s = lax.axis_index("s")
    w = c * NS + s
    ones = jnp.full((L,), 1.0, jnp.float32)
    for r, d_hbm in enumerate((d1_hbm, d2_hbm, d3_hbm)):
        pltpu.sync_copy(d_hbm.at[pl.ds(w * BPW, BPW)], dst_v)
        pltpu.sync_copy(z1d_hbm, hist_v)

        def body(j, carry):
            for k in range(EB // L):
                d16 = dst_v[j, pl.ds(k * L, L)]
                plsc.addupdate_scatter(hist_v, [d16], ones)
            return carry

        lax.fori_loop(0, BPW, body, 0)
        pltpu.sync_copy(hist_v, hist_hbm.at[r, w])


_sc_hist = pl.kernel(
    _sc_hist_body,
    compiler_params=_sc_params,
    out_type=jax.ShapeDtypeStruct((3, NW, NPAD), jnp.float32),
    mesh=_mesh,
    scratch_types=[
        pltpu.VMEM((BPW, EB), jnp.int32),
        pltpu.VMEM((NPAD,), jnp.float32),
    ],
)


def _sc_gat_logits_body(srcv_hbm, dstv_hbm, eav_hbm, hsa_hbm, hda_hbm,
                        shift_hbm, z1d_hbm,
                        ex_hbm, hist_hbm,
                        src_v, dst_v, ea_v, ex_v, hsa_v, hda_v, hist_v,
                        shift_v):
    c = lax.axis_index("c")
    s = lax.axis_index("s")
    w = c * NS + s
    pltpu.sync_copy(srcv_hbm.at[pl.ds(w * BPW, BPW)], src_v)
    pltpu.sync_copy(dstv_hbm.at[pl.ds(w * BPW, BPW)], dst_v)
    pltpu.sync_copy(eav_hbm.at[pl.ds(w * BPW, BPW)], ea_v)
    pltpu.sync_copy(hsa_hbm, hsa_v)
    pltpu.sync_copy(hda_hbm, hda_v)
    pltpu.sync_copy(z1d_hbm, hist_v)
    pltpu.sync_copy(shift_hbm, shift_v)

    def body(j, carry):
        for k in range(EB // L):
            sl = pl.ds(k * L, L)
            s16 = src_v[j, sl]
            d16 = dst_v[j, sl]
            a = (plsc.load_gather(hsa_v, [s16])
                 + plsc.load_gather(hda_v, [d16])
                 + ea_v[j, sl])
            a = jnp.maximum(a, 0.2 * a)
            ex = jnp.exp(a - shift_v[...])
            ex_v[j, sl] = ex
            plsc.addupdate_scatter(hist_v, [d16], ex)
        return carry

    lax.fori_loop(0, BPW, body, 0)
    pltpu.sync_copy(ex_v, ex_hbm.at[pl.ds(w * BPW, BPW)])
    pltpu.sync_copy(hist_v, hist_hbm.at[w])


_sc_gat_logits = pl.kernel(
    _sc_gat_logits_body,
    compiler_params=_sc_params,
    out_type=(
        jax.ShapeDtypeStruct((EROWS, EB), jnp.float32),
        jax.ShapeDtypeStruct((NW, NPAD), jnp.float32),
    ),
    mesh=_mesh,
    scratch_types=[
        pltpu.VMEM((BPW, EB), jnp.int32),
        pltpu.VMEM((BPW, EB), jnp.int32),
        pltpu.VMEM((BPW, EB), jnp.float32),
        pltpu.VMEM((BPW, EB), jnp.float32),
        pltpu.VMEM((NPAD,), jnp.float32),
        pltpu.VMEM((NPAD,), jnp.float32),
        pltpu.VMEM((NPAD,), jnp.float32),
        pltpu.VMEM((L,), jnp.float32),
    ],
)


def _sc_gat_agg_body(hs_hbm, pk_hbm, exv_hbm, deninv_hbm, z2d_hbm,
                     out_hbm,
                     pk_v, rows_v, srcb, dstb, ex_a, ex_b, dinv_v, alpha_v,
                     acc_sh, sem_ea, sem_eb, sem_s):
    c = lax.axis_index("c")
    s = lax.axis_index("s")
    w = c * NS + s
    pltpu.sync_copy(pk_hbm.at[pl.ds(w * BPW, BPW)], pk_v)
    pltpu.sync_copy(deninv_hbm, dinv_v)
    pltpu.sync_copy(z2d_hbm, rows_v)
    nblk = NPAD // EB // NS
    for b in range(nblk):
        pltpu.sync_copy(rows_v, acc_sh.at[pl.ds((s * nblk + b) * EB, EB)])
    pltpu.async_copy(exv_hbm.at[pl.ds(w * BPW, 1)], ex_a, sem_ea)
    plsc.subcore_barrier()

    def half(j, g, ex_cur, sem_cur, ex_nxt, sem_nxt):
        @pl.when(g > 0)
        def _():
            pltpu.make_async_copy(rows_v, acc_sh.at[dstb], sem_s).wait()
        _unpack(pk_v, j, srcb, dstb)
        pltpu.sync_copy(hs_hbm.at[srcb], rows_v)

        @pl.when(j + 1 < BPW)
        def _():
            pltpu.async_copy(exv_hbm.at[pl.ds(w * BPW + j + 1, 1)], ex_nxt,
                             sem_nxt)
        pltpu.make_async_copy(exv_hbm.at[pl.ds(w * BPW + j, 1)], ex_cur,
                              sem_cur).wait()
        for k in range(EB // L):
            sl = pl.ds(k * L, L)
            d16 = dstb[sl]
            alpha_v[sl] = ex_cur[0, sl] * plsc.load_gather(dinv_v, [d16])
        for r in range(EB):
            ar = plsc.load_gather(alpha_v, [jnp.full((L,), r, jnp.int32)])
            for m in range(D // L):
                sl = pl.ds(m * L, L)
                rows_v[r, sl] = rows_v[r, sl] * ar
        pltpu.async_copy(rows_v, acc_sh.at[dstb], sem_s, add=True)

    def pair(g, carry):
        j = 2 * g
        half(j, 2 * g, ex_a, sem_ea, ex_b, sem_eb)
        half(j + 1, 2 * g + 1, ex_b, sem_eb, ex_a, sem_ea)
        return carry

    lax.fori_loop(0, BPW // 2, pair, 0)
    pltpu.make_async_copy(rows_v, acc_sh.at[dstb], sem_s).wait()
    plsc.subcore_barrier()
    r0 = s * (NPAD // NS)
    pltpu.sync_copy(acc_sh.at[pl.ds(r0, NPAD // NS)],
                    out_hbm.at[c, pl.ds(r0, NPAD // NS)])


_sc_gat_agg = pl.kernel(
    _sc_gat_agg_body,
    compiler_params=_sc_params,
    out_type=jax.ShapeDtypeStruct((NC, NPAD, D), jnp.float32),
    mesh=_mesh,
    scratch_types=[
        pltpu.VMEM((BPW, EB), jnp.int32),
        pltpu.VMEM((EB, D), jnp.float32),
        pltpu.VMEM((EB,), jnp.int32),
        pltpu.VMEM((EB,), jnp.int32),
        pltpu.VMEM((1, EB), jnp.float32),
        pltpu.VMEM((1, EB), jnp.float32),
        pltpu.VMEM((NPAD,), jnp.float32),
        pltpu.VMEM((EB,), jnp.float32),
        pltpu.VMEM_SHARED((NPAD, D), jnp.float32),
        pltpu.SemaphoreType.DMA,
        pltpu.SemaphoreType.DMA,
        pltpu.SemaphoreType.DMA,
    ],
)


BT = 512
NG = NPAD // BT


def _tc_dense_body(p_ref, hist_ref, xd_ref, wl_ref, bl_ref, wr_ref, o_ref):
    deg = jnp.sum(hist_ref[...], axis=0)
    dinv = 1.0 / jnp.maximum(deg, 1.0)
    agg = (p_ref[0] + p_ref[1]) * dinv[:, None]
    o_ref[...] = jax.nn.relu(
        jnp.dot(agg, wl_ref[...], preferred_element_type=jnp.float32)
        + bl_ref[...]
        + jnp.dot(xd_ref[...], wr_ref[...], preferred_element_type=jnp.float32))


def _tc_dense(p, hist, xd, wl, bl, wr):
    return pl.pallas_call(
        _tc_dense_body,
        grid=(NG,),
        in_specs=[
            pl.BlockSpec((NC, BT, D), lambda i: (0, i, 0)),
            pl.BlockSpec((NW, BT), lambda i: (0, i)),
            pl.BlockSpec((BT, D), lambda i: (i, 0)),
            pl.BlockSpec((D, D), lambda i: (0, 0)),
            pl.BlockSpec((1, D), lambda i: (0, 0)),
            pl.BlockSpec((D, D), lambda i: (0, 0)),
        ],
        out_specs=pl.BlockSpec((BT, D), lambda i: (i, 0)),
        out_shape=jax.ShapeDtypeStruct((NPAD, D), jnp.float32),
    )(p, hist, xd, wl, bl.reshape(1, D), wr)


def _tc_final_body(p_ref, hist_ref, ph_ref, bg_ref, wl_ref, bl_ref, wr_ref,
                   wm_ref, bm_ref, o_ref):
    deg = jnp.sum(hist_ref[...], axis=0)
    dinv = 1.0 / jnp.maximum(deg, 1.0)
    agg = (p_ref[0] + p_ref[1]) * dinv[:, None]
    h = jax.nn.relu(ph_ref[0] + ph_ref[1] + bg_ref[...])
    in_x = jax.nn.relu(
        jnp.dot(agg, wl_ref[...], preferred_element_type=jnp.float32)
        + bl_ref[...]
        + jnp.dot(h, wr_ref[...], preferred_element_type=jnp.float32))
    o_ref[...] = (jnp.dot(in_x, wm_ref[...], preferred_element_type=jnp.float32)
                  + bm_ref[...])


def _tc_final(p, hist, ph, bg, wl, bl, wr, wm, bm):
    return pl.pallas_call(
        _tc_final_body,
        grid=(NG,),
        in_specs=[
            pl.BlockSpec((NC, BT, D), lambda i: (0, i, 0)),
            pl.BlockSpec((NW, BT), lambda i: (0, i)),
            pl.BlockSpec((NC, BT, D), lambda i: (0, i, 0)),
            pl.BlockSpec((1, D), lambda i: (0, 0)),
            pl.BlockSpec((D, D), lambda i: (0, 0)),
            pl.BlockSpec((1, D), lambda i: (0, 0)),
            pl.BlockSpec((D, D), lambda i: (0, 0)),
            pl.BlockSpec((D, 1), lambda i: (0, 0)),
            pl.BlockSpec((1, 1), lambda i: (0, 0)),
        ],
        out_specs=pl.BlockSpec((BT, 1), lambda i: (i, 0)),
        out_shape=jax.ShapeDtypeStruct((NPAD, 1), jnp.float32),
    )(p, hist, ph, bg.reshape(1, D), wl, bl.reshape(1, D), wr,
      wm, bm.reshape(1, 1))


def _tc_attn_pre_body(g_ref, st_ref, ws_ref, wd_ref, as_ref, ad_ref,
                      hs_ref, hsa_ref, hda_ref, m_ref):
    i = pl.program_id(0)
    hs = jnp.dot(g_ref[...], ws_ref[...], preferred_element_type=jnp.float32)
    hs_ref[...] = hs
    hsa = jnp.dot(hs, as_ref[...], preferred_element_type=jnp.float32)
    wdv = jnp.dot(wd_ref[...], ad_ref[...], preferred_element_type=jnp.float32)
    hda = jnp.dot(st_ref[...], wdv, preferred_element_type=jnp.float32)
    hsa_ref[...] = hsa
    hda_ref[...] = hda

    @pl.when(i == 0)
    def _():
        m_ref[...] = jnp.full((1, 2), -1e30, jnp.float32)

    cur = jnp.concatenate(
        [jnp.max(hsa).reshape(1, 1), jnp.max(hda).reshape(1, 1)], axis=1)
    m_ref[...] = jnp.maximum(m_ref[...], cur)


def _tc_attn_pre(g, st, ws, wd, att_s, att_d):
    return pl.pallas_call(
        _tc_attn_pre_body,
        grid=(NG,),
        in_specs=[
            pl.BlockSpec((BT, D), lambda i: (i, 0)),
            pl.BlockSpec((BT, D), lambda i: (i, 0)),
            pl.BlockSpec((D, D), lambda i: (0, 0)),
            pl.BlockSpec((D, D), lambda i: (0, 0)),
            pl.BlockSpec((D, 1), lambda i: (0, 0)),
            pl.BlockSpec((D, 1), lambda i: (0, 0)),
        ],
        out_specs=(
            pl.BlockSpec((BT, D), lambda i: (i, 0)),
            pl.BlockSpec((BT, 1), lambda i: (i, 0)),
            pl.BlockSpec((BT, 1), lambda i: (i, 0)),
            pl.BlockSpec((1, 2), lambda i: (0, 0)),
        ),
        out_shape=(
            jax.ShapeDtypeStruct((NPAD, D), jnp.float32),
            jax.ShapeDtypeStruct((NPAD, 1), jnp.float32),
            jax.ShapeDtypeStruct((NPAD, 1), jnp.float32),
            jax.ShapeDtypeStruct((1, 2), jnp.float32),
        ),
    )(g, st, ws, wd, att_s.reshape(D, 1), att_d.reshape(D, 1))


BTE = 2000


def _tc_ea_body(ea_ref, we_ref, o_ref, m_ref):
    i = pl.program_id(0)
    v = jnp.sum(ea_ref[...] * we_ref[...], axis=1, keepdims=True)
    o_ref[...] = v

    @pl.when(i == 0)
    def _():
        m_ref[...] = jnp.full((1, 1), -1e30, jnp.float32)

    m_ref[...] = jnp.maximum(m_ref[...], jnp.max(v).reshape(1, 1))


def _tc_ea(edge_attr, wg_e, att_e):
    we = jnp.dot(wg_e, att_e.reshape(D, 1),
                 preferred_element_type=jnp.float32)
    return pl.pallas_call(
        _tc_ea_body,
        grid=(E // BTE,),
        in_specs=[
            pl.BlockSpec((BTE, DE), lambda i: (i, 0)),
            pl.BlockSpec((1, DE), lambda i: (0, 0)),
        ],
        out_specs=(
            pl.BlockSpec((BTE, 1), lambda i: (i, 0)),
            pl.BlockSpec((1, 1), lambda i: (0, 0)),
        ),
        out_shape=(
            jax.ShapeDtypeStruct((E, 1), jnp.float32),
            jax.ShapeDtypeStruct((1, 1), jnp.float32),
        ),
    )(edge_attr, we.reshape(1, DE))


def _tc_deninv_body(hist_ref, o_ref):
    den = jnp.sum(hist_ref[...], axis=0, keepdims=True)
    o_ref[...] = 1.0 / jnp.maximum(den, 1e-16)


def _tc_deninv(hist):
    return pl.pallas_call(
        _tc_deninv_body,
        out_shape=jax.ShapeDtypeStruct((1, NPAD), jnp.float32),
    )(hist)


def _pad_ei(ei):
    src = jnp.concatenate(
        [ei[0], jnp.zeros((EPAD - E,), ei.dtype)]).astype(jnp.int32)
    dst = jnp.concatenate(
        [ei[1], jnp.full((EPAD - E,), DUMP, ei.dtype)]).astype(jnp.int32)
    pk = jnp.bitwise_or(src, jnp.left_shift(dst, SHIFT))
    return (src.reshape(EROWS, EB), dst.reshape(EROWS, EB),
            pk.reshape(EROWS, EB))


def _pad_x(x):
    return jnp.concatenate([x, jnp.zeros((NPAD - N, D), x.dtype)], axis=0)


def kernel(x_game, x_state, edge_attr, Wl, bl, Wr, Wg_s, Wg_d, Wg_e,
           att_s, att_d, att_e, bg, Wm, bm, ei_gg, ei_ss, ei_hist, ei_in):
    z2d = jnp.zeros((EB, D), jnp.float32)
    z2ds = jnp.zeros((EBS, D), jnp.float32)
    z1d = jnp.zeros((NPAD,), jnp.float32)
    _, dgg, pgg = _pad_ei(ei_gg)
    _, dss, pss = _pad_ei(ei_ss)
    shh, dhh, phh = _pad_ei(ei_hist)
    _, din, pin_ = _pad_ei(ei_in)
    xg = _pad_x(x_game)
    xs = _pad_x(x_state)

    hist3 = _sc_hist(dgg, dss, din, z1d)
    hist_gg, hist_ss, hist_in = hist3[0], hist3[1], hist3[2]

    p = _sc_seg(xg, pgg, z2ds)
    g = _tc_dense(p, hist_gg, xg, Wl[0], bl[0], Wr[0])
    p = _sc_seg(g, pgg, z2ds)
    g = _tc_dense(p, hist_gg, g, Wl[1], bl[1], Wr[1])

    p = _sc_seg(xs, pss, z2ds)
    st = _tc_dense(p, hist_ss, xs, Wl[2], bl[2], Wr[2])
    p = _sc_seg(st, pss, z2ds)
    st = _tc_dense(p, hist_ss, st, Wl[3], bl[3], Wr[3])

    hs, hsa, hda, m12 = _tc_attn_pre(g, st, Wg_s, Wg_d, att_s, att_d)
    ea, m3 = _tc_ea(edge_attr, Wg_e, att_e)
    shift = jnp.maximum(m12[0, 0] + m12[0, 1] + m3[0, 0], 0.0)
    shift16 = jnp.broadcast_to(shift, (L,))
    eav = jnp.concatenate(
        [ea.reshape(E), jnp.zeros((EPAD - E,), jnp.float32)]).reshape(EROWS, EB)
    ex, hist_den = _sc_gat_logits(shh, dhh, eav, hsa.reshape(NPAD),
                                  hda.reshape(NPAD), shift16, z1d)
    deninv = _tc_deninv(hist_den).reshape(NPAD)
    ph = _sc_gat_agg(hs, phh, ex, deninv, z2d)

    p = _sc_seg(g, pin_, z2ds)
    out = _tc_final(p, hist_in, ph, bg, Wl[4], bl[4], Wr[4], Wm, bm)
    return out[:N]


# speedup vs baseline: 1.2361x; 1.1118x over previous
import functools

import jax
import jax.numpy as jnp
from jax import lax
from jax.experimental import pallas as pl
from jax.experimental.pallas import tpu as pltpu
from jax.experimental.pallas import tpu_sc as plsc

N = 10000
E = 320000
D = 128
DE = 16

NC = 2
NS = 16
NW = NC * NS
L = 16

NPAD = 10240
EB = 128
EW = 10240
BPW = EW // EB
EPAD = EW * NW
EROWS = EPAD // EB
DUMP = NPAD - 1
SHIFT = 14

_mesh = plsc.VectorSubcoreMesh(
    core_axis_name="c", subcore_axis_name="s", num_cores=NC, num_subcores=NS)
_sc_params = pltpu.CompilerParams(needs_layout_passes=False)


def _unpack(pk_v, j, srcb, dstb):
    for k in range(EB // L):
        sl = pl.ds(k * L, L)
        p16 = pk_v[j, sl]
        srcb[sl] = jnp.bitwise_and(p16, (1 << SHIFT) - 1)
        dstb[sl] = lax.shift_right_logical(p16, SHIFT)


EBS = 64
NBS = EW // EBS


def _sc_seg_body(x_hbm, pk_hbm, z2d_hbm,
                 out_hbm,
                 pk_v, r0, r1, r2, r3, s0, s1, s2, s3, d0, d1, d2, d3, acc_sh,
                 g0, g1, g2, g3, t0, t1, t2, t3):
    c = lax.axis_index("c")
    s = lax.axis_index("s")
    w = c * NS + s
    rows = (r0, r1, r2, r3)
    sb = (s0, s1, s2, s3)
    db = (d0, d1, d2, d3)
    gsem = (g0, g1, g2, g3)
    ssem = (t0, t1, t2, t3)
    pltpu.sync_copy(pk_hbm.at[pl.ds(w * BPW, BPW)], pk_v)
    pltpu.sync_copy(z2d_hbm, r0)
    nblk = NPAD // EBS // NS
    for b in range(nblk):
        pltpu.sync_copy(r0, acc_sh.at[pl.ds((s * nblk + b) * EBS, EBS)])

    def unpack64(j, srcb, dstb):
        row = lax.shift_right_logical(j, 1)
        base = jnp.bitwise_and(j, 1) * EBS
        for k in range(EBS // L):
            sl = pl.ds(base + k * L, L)
            p16 = pk_v[row, sl]
            srcb[pl.ds(k * L, L)] = jnp.bitwise_and(p16, (1 << SHIFT) - 1)
            dstb[pl.ds(k * L, L)] = lax.shift_right_logical(p16, SHIFT)

    for q in range(3):
        unpack64(jnp.int32(q), sb[q], db[q])
        pltpu.async_copy(x_hbm.at[sb[q]], rows[q], gsem[q])
    plsc.subcore_barrier()

    def quad(g, carry):
        for q in range(4):
            j = 4 * g + q
            snx = (q + 3) % 4
            pltpu.make_async_copy(x_hbm.at[sb[q]], rows[q], gsem[q]).wait()

            @pl.when(j > 0)
            def _():
                pltpu.make_async_copy(rows[snx], acc_sh.at[db[snx]],
                                      ssem[snx]).wait()

            pltpu.async_copy(rows[q], acc_sh.at[db[q]], ssem[q], add=True)

            @pl.when(j + 3 < NBS)
            def _():
                unpack64(j + 3, sb[snx], db[snx])
                pltpu.async_copy(x_hbm.at[sb[snx]], rows[snx], gsem[snx])
        return carry

    lax.fori_loop(0, NBS // 4, quad, 0)
    pltpu.make_async_copy(rows[3], acc_sh.at[db[3]], ssem[3]).wait()
    plsc.subcore_barrier()
    rr = s * (NPAD // NS)
    pltpu.sync_copy(acc_sh.at[pl.ds(rr, NPAD // NS)],
                    out_hbm.at[c, pl.ds(rr, NPAD // NS)])


_sc_seg = pl.kernel(
    _sc_seg_body,
    compiler_params=_sc_params,
    out_type=jax.ShapeDtypeStruct((NC, NPAD, D), jnp.float32),
    mesh=_mesh,
    scratch_types=(
        [pltpu.VMEM((BPW, EB), jnp.int32)]
        + [pltpu.VMEM((EBS, D), jnp.float32)] * 4
        + [pltpu.VMEM((EBS,), jnp.int32)] * 8
        + [pltpu.VMEM_SHARED((NPAD, D), jnp.float32)]
        + [pltpu.SemaphoreType.DMA] * 8
    ),
)


def _sc_hist_body(d1_hbm, d2_hbm, d3_hbm, z1d_hbm,
                  hist_hbm,
                  dst_v, hist_v):
    c = lax.axis_index("c")
    s = lax.axis_index("s")
    w = c * NS + s
    ones = jnp.full((L,), 1.0, jnp.float32)
    for r, d_hbm in enumerate((d1_hbm, d2_hbm, d3_hbm)):
        pltpu.sync_copy(d_hbm.at[pl.ds(w * BPW, BPW)], dst_v)
        pltpu.sync_copy(z1d_hbm, hist_v)

        def body(j, carry):
            for k in range(EB // L):
                d16 = dst_v[j, pl.ds(k * L, L)]
                plsc.addupdate_scatter(hist_v, [d16], ones)
            return carry

        lax.fori_loop(0, BPW, body, 0)
        pltpu.sync_copy(hist_v, hist_hbm.at[r, w])


_sc_hist = pl.kernel(
    _sc_hist_body,
    compiler_params=_sc_params,
    out_type=jax.ShapeDtypeStruct((3, NW, NPAD), jnp.float32),
    mesh=_mesh,
    scratch_types=[
        pltpu.VMEM((BPW, EB), jnp.int32),
        pltpu.VMEM((NPAD,), jnp.float32),
    ],
)


def _sc_gat_logits_body(srcv_hbm, dstv_hbm, eav_hbm, hsa_hbm, hda_hbm,
                        shift_hbm, z1d_hbm,
                        ex_hbm, hist_hbm,
                        src_v, dst_v, ea_v, ex_v, hsa_v, hda_v, hist_v,
                        shift_v):
    c = lax.axis_index("c")
    s = lax.axis_index("s")
    w = c * NS + s
    pltpu.sync_copy(srcv_hbm.at[pl.ds(w * BPW, BPW)], src_v)
    pltpu.sync_copy(dstv_hbm.at[pl.ds(w * BPW, BPW)], dst_v)
    pltpu.sync_copy(eav_hbm.at[pl.ds(w * BPW, BPW)], ea_v)
    pltpu.sync_copy(hsa_hbm, hsa_v)
    pltpu.sync_copy(hda_hbm, hda_v)
    pltpu.sync_copy(z1d_hbm, hist_v)
    pltpu.sync_copy(shift_hbm, shift_v)

    def body(j, carry):
        for k in range(EB // L):
            sl = pl.ds(k * L, L)
            s16 = src_v[j, sl]
            d16 = dst_v[j, sl]
            a = (plsc.load_gather(hsa_v, [s16])
                 + plsc.load_gather(hda_v, [d16])
                 + ea_v[j, sl])
            a = jnp.maximum(a, 0.2 * a)
            ex = jnp.exp(a - shift_v[...])
            ex_v[j, sl] = ex
            plsc.addupdate_scatter(hist_v, [d16], ex)
        return carry

    lax.fori_loop(0, BPW, body, 0)
    pltpu.sync_copy(ex_v, ex_hbm.at[pl.ds(w * BPW, BPW)])
    pltpu.sync_copy(hist_v, hist_hbm.at[w])


_sc_gat_logits = pl.kernel(
    _sc_gat_logits_body,
    compiler_params=_sc_params,
    out_type=(
        jax.ShapeDtypeStruct((EROWS, EB), jnp.float32),
        jax.ShapeDtypeStruct((NW, NPAD), jnp.float32),
    ),
    mesh=_mesh,
    scratch_types=[
        pltpu.VMEM((BPW, EB), jnp.int32),
        pltpu.VMEM((BPW, EB), jnp.int32),
        pltpu.VMEM((BPW, EB), jnp.float32),
        pltpu.VMEM((BPW, EB), jnp.float32),
        pltpu.VMEM((NPAD,), jnp.float32),
        pltpu.VMEM((NPAD,), jnp.float32),
        pltpu.VMEM((NPAD,), jnp.float32),
        pltpu.VMEM((L,), jnp.float32),
    ],
)


def _sc_gat_agg_body(hs_hbm, pk_hbm, exv_hbm, deninv_hbm, z2d_hbm,
                     out_hbm,
                     pk_v, rows_a, rows_b, sba, sbb, dba, dbb, ex_a, ex_b,
                     dinv_v, alpha_v,
                     acc_sh, ga, gb, ea, eb, ta, tb):
    c = lax.axis_index("c")
    s = lax.axis_index("s")
    w = c * NS + s
    pltpu.sync_copy(pk_hbm.at[pl.ds(w * BPW, BPW)], pk_v)
    pltpu.sync_copy(deninv_hbm, dinv_v)
    pltpu.sync_copy(z2d_hbm, rows_a)
    nblk = NPAD // EBS // NS
    for b in range(nblk):
        pltpu.sync_copy(rows_a, acc_sh.at[pl.ds((s * nblk + b) * EBS, EBS)])

    def unpack64(j, srcb, dstb):
        row = lax.shift_right_logical(j, 1)
        base = jnp.bitwise_and(j, 1) * EBS
        for k in range(EBS // L):
            sl = pl.ds(base + k * L, L)
            p16 = pk_v[row, sl]
            srcb[pl.ds(k * L, L)] = jnp.bitwise_and(p16, (1 << SHIFT) - 1)
            dstb[pl.ds(k * L, L)] = lax.shift_right_logical(p16, SHIFT)

    def exsrc(j):
        row = lax.shift_right_logical(w * NBS + j, 1)
        base = jnp.bitwise_and(j, 1) * EBS
        return exv_hbm.at[row, pl.ds(base, EBS)]

    unpack64(jnp.int32(0), sba, dba)
    pltpu.async_copy(hs_hbm.at[sba], rows_a, ga)
    pltpu.async_copy(exsrc(jnp.int32(0)), ex_a, ea)
    plsc.subcore_barrier()

    def half(j, drain_y, rows_x, sbx, dbx, ex_x, gx, ex_sx, tx,
             rows_y, sby, dby, ex_y, gy, ex_sy, ty):
        @pl.when(drain_y)
        def _():
            pltpu.make_async_copy(rows_y, acc_sh.at[dby], ty).wait()

        @pl.when(j + 1 < NBS)
        def _():
            unpack64(j + 1, sby, dby)
            pltpu.async_copy(hs_hbm.at[sby], rows_y, gy)
            pltpu.async_copy(exsrc(j + 1), ex_y, ex_sy)
        pltpu.make_async_copy(hs_hbm.at[sbx], rows_x, gx).wait()
        pltpu.make_async_copy(exsrc(j), ex_x, ex_sx).wait()
        for k in range(EBS // L):
            sl = pl.ds(k * L, L)
            d16 = dbx[sl]
            alpha_v[sl] = ex_x[sl] * plsc.load_gather(dinv_v, [d16])
        for r in range(EBS):
            ar = plsc.load_gather(alpha_v, [jnp.full((L,), r, jnp.int32)])
            for m in range(D // L):
                sl = pl.ds(m * L, L)
                rows_x[r, sl] = rows_x[r, sl] * ar
        pltpu.async_copy(rows_x, acc_sh.at[dbx], tx, add=True)

    def pair(g, carry):
        j = 2 * g
        half(j, g > 0, rows_a, sba, dba, ex_a, ga, ea, ta,
             rows_b, sbb, dbb, ex_b, gb, eb, tb)
        half(j + 1, g >= 0, rows_b, sbb, dbb, ex_b, gb, eb, tb,
             rows_a, sba, dba, ex_a, ga, ea, ta)
        return carry

    lax.fori_loop(0, NBS // 2, pair, 0)
    pltpu.make_async_copy(rows_b, acc_sh.at[dbb], tb).wait()
    plsc.subcore_barrier()
    rr = s * (NPAD // NS)
    pltpu.sync_copy(acc_sh.at[pl.ds(rr, NPAD // NS)],
                    out_hbm.at[c, pl.ds(rr, NPAD // NS)])


_sc_gat_agg = pl.kernel(
    _sc_gat_agg_body,
    compiler_params=_sc_params,
    out_type=jax.ShapeDtypeStruct((NC, NPAD, D), jnp.float32),
    mesh=_mesh,
    scratch_types=(
        [pltpu.VMEM((BPW, EB), jnp.int32)]
        + [pltpu.VMEM((EBS, D), jnp.float32)] * 2
        + [pltpu.VMEM((EBS,), jnp.int32)] * 4
        + [pltpu.VMEM((EBS,), jnp.float32)] * 2
        + [pltpu.VMEM((NPAD,), jnp.float32)]
        + [pltpu.VMEM((EBS,), jnp.float32)]
        + [pltpu.VMEM_SHARED((NPAD, D), jnp.float32)]
        + [pltpu.SemaphoreType.DMA] * 6
    ),
)


BT = 1024
NG = NPAD // BT


def _tc_dense_body(p_ref, hist_ref, xd_ref, wl_ref, bl_ref, wr_ref, o_ref):
    deg = jnp.sum(hist_ref[...], axis=0)
    dinv = 1.0 / jnp.maximum(deg, 1.0)
    agg = (p_ref[0] + p_ref[1]) * dinv[:, None]
    o_ref[...] = jax.nn.relu(
        jnp.dot(agg, wl_ref[...], preferred_element_type=jnp.float32)
        + bl_ref[...]
        + jnp.dot(xd_ref[...], wr_ref[...], preferred_element_type=jnp.float32))


def _tc_dense(p, hist, xd, wl, bl, wr):
    return pl.pallas_call(
        _tc_dense_body,
        grid=(NG,),
        in_specs=[
            pl.BlockSpec((NC, BT, D), lambda i: (0, i, 0)),
            pl.BlockSpec((NW, BT), lambda i: (0, i)),
            pl.BlockSpec((BT, D), lambda i: (i, 0)),
            pl.BlockSpec((D, D), lambda i: (0, 0)),
            pl.BlockSpec((1, D), lambda i: (0, 0)),
            pl.BlockSpec((D, D), lambda i: (0, 0)),
        ],
        out_specs=pl.BlockSpec((BT, D), lambda i: (i, 0)),
        out_shape=jax.ShapeDtypeStruct((NPAD, D), jnp.float32),
    )(p, hist, xd, wl, bl.reshape(1, D), wr)


def _tc_final_body(p_ref, hist_ref, ph_ref, bg_ref, wl_ref, bl_ref, wr_ref,
                   wm_ref, bm_ref, o_ref):
    deg = jnp.sum(hist_ref[...], axis=0)
    dinv = 1.0 / jnp.maximum(deg, 1.0)
    agg = (p_ref[0] + p_ref[1]) * dinv[:, None]
    h = jax.nn.relu(ph_ref[0] + ph_ref[1] + bg_ref[...])
    in_x = jax.nn.relu(
        jnp.dot(agg, wl_ref[...], preferred_element_type=jnp.float32)
        + bl_ref[...]
        + jnp.dot(h, wr_ref[...], preferred_element_type=jnp.float32))
    o_ref[...] = (jnp.dot(in_x, wm_ref[...], preferred_element_type=jnp.float32)
                  + bm_ref[...])


def _tc_final(p, hist, ph, bg, wl, bl, wr, wm, bm):
    return pl.pallas_call(
        _tc_final_body,
        grid=(NG,),
        in_specs=[
            pl.BlockSpec((NC, BT, D), lambda i: (0, i, 0)),
            pl.BlockSpec((NW, BT), lambda i: (0, i)),
            pl.BlockSpec((NC, BT, D), lambda i: (0, i, 0)),
            pl.BlockSpec((1, D), lambda i: (0, 0)),
            pl.BlockSpec((D, D), lambda i: (0, 0)),
            pl.BlockSpec((1, D), lambda i: (0, 0)),
            pl.BlockSpec((D, D), lambda i: (0, 0)),
            pl.BlockSpec((D, 1), lambda i: (0, 0)),
            pl.BlockSpec((1, 1), lambda i: (0, 0)),
        ],
        out_specs=pl.BlockSpec((BT, 1), lambda i: (i, 0)),
        out_shape=jax.ShapeDtypeStruct((NPAD, 1), jnp.float32),
    )(p, hist, ph, bg.reshape(1, D), wl, bl.reshape(1, D), wr,
      wm, bm.reshape(1, 1))


def _tc_attn_pre_body(g_ref, st_ref, ws_ref, wd_ref, as_ref, ad_ref,
                      hs_ref, hsa_ref, hda_ref, m_ref):
    i = pl.program_id(0)
    hs = jnp.dot(g_ref[...], ws_ref[...], preferred_element_type=jnp.float32)
    hs_ref[...] = hs
    hsa = jnp.dot(hs, as_ref[...], preferred_element_type=jnp.float32)
    wdv = jnp.dot(wd_ref[...], ad_ref[...], preferred_element_type=jnp.float32)
    hda = jnp.dot(st_ref[...], wdv, preferred_element_type=jnp.float32)
    hsa_ref[...] = hsa.reshape(BT // 128, 128)
    hda_ref[...] = hda.reshape(BT // 128, 128)

    @pl.when(i == 0)
    def _():
        m_ref[...] = jnp.full((1, 2), -1e30, jnp.float32)

    cur = jnp.concatenate(
        [jnp.max(hsa).reshape(1, 1), jnp.max(hda).reshape(1, 1)], axis=1)
    m_ref[...] = jnp.maximum(m_ref[...], cur)


def _tc_attn_pre(g, st, ws, wd, att_s, att_d):
    return pl.pallas_call(
        _tc_attn_pre_body,
        grid=(NG,),
        in_specs=[
            pl.BlockSpec((BT, D), lambda i: (i, 0)),
            pl.BlockSpec((BT, D), lambda i: (i, 0)),
            pl.BlockSpec((D, D), lambda i: (0, 0)),
            pl.BlockSpec((D, D), lambda i: (0, 0)),
            pl.BlockSpec((D, 1), lambda i: (0, 0)),
            pl.BlockSpec((D, 1), lambda i: (0, 0)),
        ],
        out_specs=(
            pl.BlockSpec((BT, D), lambda i: (i, 0)),
            pl.BlockSpec((BT // 128, 128), lambda i: (i, 0)),
            pl.BlockSpec((BT // 128, 128), lambda i: (i, 0)),
            pl.BlockSpec((1, 2), lambda i: (0, 0)),
        ),
        out_shape=(
            jax.ShapeDtypeStruct((NPAD, D), jnp.float32),
            jax.ShapeDtypeStruct((NPAD // 128, 128), jnp.float32),
            jax.ShapeDtypeStruct((NPAD // 128, 128), jnp.float32),
            jax.ShapeDtypeStruct((1, 2), jnp.float32),
        ),
    )(g, st, ws, wd, att_s.reshape(D, 1), att_d.reshape(D, 1))


EA_BT = 2048


def _tc_ea_body(ea_ref, wmat_ref, o_ref, m_ref):
    i = pl.program_id(0)
    o8 = jnp.dot(ea_ref[...], wmat_ref[...],
                 preferred_element_type=jnp.float32)
    o_ref[...] = o8

    @pl.when(i == 0)
    def _():
        m_ref[...] = jnp.full((1, 1), -1e30, jnp.float32)

    m_ref[...] = jnp.maximum(m_ref[...], jnp.max(o8).reshape(1, 1))


def _tc_ea(edge_attr, wg_e, att_e):
    we = jnp.dot(wg_e, att_e.reshape(D, 1),
                 preferred_element_type=jnp.float32).reshape(DE)
    wmat = (jnp.eye(8, dtype=jnp.float32)[:, None, :]
            * we[None, :, None]).reshape(128, 8)
    ea2 = jnp.concatenate(
        [edge_attr.reshape(E // 8, 128),
         jnp.zeros(((EPAD - E) // 8, 128), jnp.float32)], axis=0)
    o8, m3 = pl.pallas_call(
        _tc_ea_body,
        grid=(EPAD // 8 // EA_BT,),
        in_specs=[
            pl.BlockSpec((EA_BT, 128), lambda i: (i, 0)),
            pl.BlockSpec((128, 8), lambda i: (0, 0)),
        ],
        out_specs=(
            pl.BlockSpec((EA_BT, 8), lambda i: (i, 0)),
            pl.BlockSpec((1, 1), lambda i: (0, 0)),
        ),
        out_shape=(
            jax.ShapeDtypeStruct((EPAD // 8, 8), jnp.float32),
            jax.ShapeDtypeStruct((1, 1), jnp.float32),
        ),
    )(ea2, wmat)
    return o8.reshape(EROWS, EB), m3


def _tc_deninv_body(hist_ref, o_ref):
    den = jnp.sum(hist_ref[...], axis=0, keepdims=True)
    o_ref[...] = 1.0 / jnp.maximum(den, 1e-16)


def _tc_deninv(hist):
    return pl.pallas_call(
        _tc_deninv_body,
        out_shape=jax.ShapeDtypeStruct((1, NPAD), jnp.float32),
    )(hist)


def _pad_ei(ei):
    src = jnp.concatenate(
        [ei[0], jnp.zeros((EPAD - E,), ei.dtype)]).astype(jnp.int32)
    dst = jnp.concatenate(
        [ei[1], jnp.full((EPAD - E,), DUMP, ei.dtype)]).astype(jnp.int32)
    pk = jnp.bitwise_or(src, jnp.left_shift(dst, SHIFT))
    return (src.reshape(EROWS, EB), dst.reshape(EROWS, EB),
            pk.reshape(EROWS, EB))


def _pad_x(x):
    return jnp.concatenate([x, jnp.zeros((NPAD - N, D), x.dtype)], axis=0)


def kernel(x_game, x_state, edge_attr, Wl, bl, Wr, Wg_s, Wg_d, Wg_e,
           att_s, att_d, att_e, bg, Wm, bm, ei_gg, ei_ss, ei_hist, ei_in):
    z2d = jnp.zeros((EB, D), jnp.float32)
    z2ds = jnp.zeros((EBS, D), jnp.float32)
    z1d = jnp.zeros((NPAD,), jnp.float32)
    _, dgg, pgg = _pad_ei(ei_gg)
    _, dss, pss = _pad_ei(ei_ss)
    shh, dhh, phh = _pad_ei(ei_hist)
    _, din, pin_ = _pad_ei(ei_in)
    xg = _pad_x(x_game)
    xs = _pad_x(x_state)

    hist3 = _sc_hist(dgg, dss, din, z1d)
    hist_gg, hist_ss, hist_in = hist3[0], hist3[1], hist3[2]

    p = _sc_seg(xg, pgg, z2ds)
    g = _tc_dense(p, hist_gg, xg, Wl[0], bl[0], Wr[0])
    p = _sc_seg(g, pgg, z2ds)
    g = _tc_dense(p, hist_gg, g, Wl[1], bl[1], Wr[1])

    p = _sc_seg(xs, pss, z2ds)
    st = _tc_dense(p, hist_ss, xs, Wl[2], bl[2], Wr[2])
    p = _sc_seg(st, pss, z2ds)
    st = _tc_dense(p, hist_ss, st, Wl[3], bl[3], Wr[3])

    hs, hsa, hda, m12 = _tc_attn_pre(g, st, Wg_s, Wg_d, att_s, att_d)
    eav, m3 = _tc_ea(edge_attr, Wg_e, att_e)
    shift = jnp.maximum(m12[0, 0] + m12[0, 1] + m3[0, 0], 0.0)
    shift16 = jnp.broadcast_to(shift, (L,))
    ex, hist_den = _sc_gat_logits(shh, dhh, eav, hsa.reshape(NPAD),
                                  hda.reshape(NPAD), shift16, z1d)
    deninv = _tc_deninv(hist_den).reshape(NPAD)
    ph = _sc_gat_agg(hs, phh, ex, deninv, z2ds)

    p = _sc_seg(g, pin_, z2ds)
    out = _tc_final(p, hist_in, ph, bg, Wl[4], bl[4], Wr[4], Wm, bm)
    return out[:N]


# speedup vs baseline: 1.2373x; 1.0010x over previous
import functools

import jax
import jax.numpy as jnp
from jax import lax
from jax.experimental import pallas as pl
from jax.experimental.pallas import tpu as pltpu
from jax.experimental.pallas import tpu_sc as plsc

N = 10000
E = 320000
D = 128
DE = 16

NC = 2
NS = 16
NW = NC * NS
L = 16

NPAD = 10240
EB = 128
EW = 10240
BPW = EW // EB
EPAD = EW * NW
EROWS = EPAD // EB
DUMP = NPAD - 1
SHIFT = 14

_mesh = plsc.VectorSubcoreMesh(
    core_axis_name="c", subcore_axis_name="s", num_cores=NC, num_subcores=NS)
_sc_params = pltpu.CompilerParams(needs_layout_passes=False)


def _unpack(pk_v, j, srcb, dstb):
    for k in range(EB // L):
        sl = pl.ds(k * L, L)
        p16 = pk_v[j, sl]
        srcb[sl] = jnp.bitwise_and(p16, (1 << SHIFT) - 1)
        dstb[sl] = lax.shift_right_logical(p16, SHIFT)


EBS = 64
NBS = EW // EBS


def _sc_seg_body(x_hbm, pk_hbm, z2d_hbm,
                 out_hbm,
                 pk_v, r0, r1, r2, r3, s0, s1, s2, s3, d0, d1, d2, d3, acc_sh,
                 g0, g1, g2, g3, t0, t1, t2, t3):
    c = lax.axis_index("c")
    s = lax.axis_index("s")
    w = c * NS + s
    rows = (r0, r1, r2, r3)
    sb = (s0, s1, s2, s3)
    db = (d0, d1, d2, d3)
    gsem = (g0, g1, g2, g3)
    ssem = (t0, t1, t2, t3)
    pltpu.sync_copy(pk_hbm.at[pl.ds(w * BPW, BPW)], pk_v)
    pltpu.sync_copy(z2d_hbm, r3)
    nblk = NPAD // EBS // NS

    def unpack64(j, srcb, dstb):
        row = lax.shift_right_logical(j, 1)
        base = jnp.bitwise_and(j, 1) * EBS
        for k in range(EBS // L):
            sl = pl.ds(base + k * L, L)
            p16 = pk_v[row, sl]
            srcb[pl.ds(k * L, L)] = jnp.bitwise_and(p16, (1 << SHIFT) - 1)
            dstb[pl.ds(k * L, L)] = lax.shift_right_logical(p16, SHIFT)

    for q in range(3):
        unpack64(jnp.int32(q), sb[q], db[q])
        pltpu.async_copy(x_hbm.at[sb[q]], rows[q], gsem[q])
    for b in range(nblk):
        pltpu.async_copy(r3, acc_sh.at[pl.ds((s * nblk + b) * EBS, EBS)], t3)
    for b in range(nblk):
        pltpu.make_async_copy(
            r3, acc_sh.at[pl.ds((s * nblk + b) * EBS, EBS)], t3).wait()
    plsc.subcore_barrier()

    def quad(g, carry):
        for q in range(4):
            j = 4 * g + q
            snx = (q + 3) % 4
            pltpu.make_async_copy(x_hbm.at[sb[q]], rows[q], gsem[q]).wait()

            @pl.when(j > 0)
            def _():
                pltpu.make_async_copy(rows[snx], acc_sh.at[db[snx]],
                                      ssem[snx]).wait()

            pltpu.async_copy(rows[q], acc_sh.at[db[q]], ssem[q], add=True)

            @pl.when(j + 3 < NBS)
            def _():
                unpack64(j + 3, sb[snx], db[snx])
                pltpu.async_copy(x_hbm.at[sb[snx]], rows[snx], gsem[snx])
        return carry

    lax.fori_loop(0, NBS // 4, quad, 0)
    pltpu.make_async_copy(rows[3], acc_sh.at[db[3]], ssem[3]).wait()
    plsc.subcore_barrier()
    rr = s * (NPAD // NS)
    pltpu.sync_copy(acc_sh.at[pl.ds(rr, NPAD // NS)],
                    out_hbm.at[c, pl.ds(rr, NPAD // NS)])


_sc_seg = pl.kernel(
    _sc_seg_body,
    compiler_params=_sc_params,
    out_type=jax.ShapeDtypeStruct((NC, NPAD, D), jnp.float32),
    mesh=_mesh,
    scratch_types=(
        [pltpu.VMEM((BPW, EB), jnp.int32)]
        + [pltpu.VMEM((EBS, D), jnp.float32)] * 4
        + [pltpu.VMEM((EBS,), jnp.int32)] * 8
        + [pltpu.VMEM_SHARED((NPAD, D), jnp.float32)]
        + [pltpu.SemaphoreType.DMA] * 8
    ),
)


def _sc_hist_body(d1_hbm, d2_hbm, d3_hbm, z1d_hbm,
                  hist_hbm,
                  dst_v, hist_v):
    c = lax.axis_index("c")
    s = lax.axis_index("s")
    w = c * NS + s
    ones = jnp.full((L,), 1.0, jnp.float32)
    for r, d_hbm in enumerate((d1_hbm, d2_hbm, d3_hbm)):
        pltpu.sync_copy(d_hbm.at[pl.ds(w * BPW, BPW)], dst_v)
        pltpu.sync_copy(z1d_hbm, hist_v)

        def body(j, carry):
            for k in range(EB // L):
                d16 = dst_v[j, pl.ds(k * L, L)]
                plsc.addupdate_scatter(hist_v, [d16], ones)
            return carry

        lax.fori_loop(0, BPW, body, 0)
        pltpu.sync_copy(hist_v, hist_hbm.at[r, w])


_sc_hist = pl.kernel(
    _sc_hist_body,
    compiler_params=_sc_params,
    out_type=jax.ShapeDtypeStruct((3, NW, NPAD), jnp.float32),
    mesh=_mesh,
    scratch_types=[
        pltpu.VMEM((BPW, EB), jnp.int32),
        pltpu.VMEM((NPAD,), jnp.float32),
    ],
)


def _sc_gat_logits_body(srcv_hbm, dstv_hbm, eav_hbm, hsa_hbm, hda_hbm,
                        shift_hbm, z1d_hbm,
                        ex_hbm, hist_hbm,
                        src_v, dst_v, ea_v, ex_v, hsa_v, hda_v, hist_v,
                        shift_v):
    c = lax.axis_index("c")
    s = lax.axis_index("s")
    w = c * NS + s
    pltpu.sync_copy(srcv_hbm.at[pl.ds(w * BPW, BPW)], src_v)
    pltpu.sync_copy(dstv_hbm.at[pl.ds(w * BPW, BPW)], dst_v)
    pltpu.sync_copy(eav_hbm.at[pl.ds(w * BPW, BPW)], ea_v)
    pltpu.sync_copy(hsa_hbm, hsa_v)
    pltpu.sync_copy(hda_hbm, hda_v)
    pltpu.sync_copy(z1d_hbm, hist_v)
    pltpu.sync_copy(shift_hbm, shift_v)

    def body(j, carry):
        for k in range(EB // L):
            sl = pl.ds(k * L, L)
            s16 = src_v[j, sl]
            d16 = dst_v[j, sl]
            a = (plsc.load_gather(hsa_v, [s16])
                 + plsc.load_gather(hda_v, [d16])
                 + ea_v[j, sl])
            a = jnp.maximum(a, 0.2 * a)
            ex = jnp.exp(a - shift_v[...])
            ex_v[j, sl] = ex
            plsc.addupdate_scatter(hist_v, [d16], ex)
        return carry

    lax.fori_loop(0, BPW, body, 0)
    pltpu.sync_copy(ex_v, ex_hbm.at[pl.ds(w * BPW, BPW)])
    pltpu.sync_copy(hist_v, hist_hbm.at[w])


_sc_gat_logits = pl.kernel(
    _sc_gat_logits_body,
    compiler_params=_sc_params,
    out_type=(
        jax.ShapeDtypeStruct((EROWS, EB), jnp.float32),
        jax.ShapeDtypeStruct((NW, NPAD), jnp.float32),
    ),
    mesh=_mesh,
    scratch_types=[
        pltpu.VMEM((BPW, EB), jnp.int32),
        pltpu.VMEM((BPW, EB), jnp.int32),
        pltpu.VMEM((BPW, EB), jnp.float32),
        pltpu.VMEM((BPW, EB), jnp.float32),
        pltpu.VMEM((NPAD,), jnp.float32),
        pltpu.VMEM((NPAD,), jnp.float32),
        pltpu.VMEM((NPAD,), jnp.float32),
        pltpu.VMEM((L,), jnp.float32),
    ],
)


def _sc_gat_agg_body(hs_hbm, pk_hbm, exv_hbm, deninv_hbm, z2d_hbm,
                     out_hbm,
                     pk_v, rows_a, rows_b, sba, sbb, dba, dbb, ex_a, ex_b,
                     dinv_v, alpha_v,
                     acc_sh, ga, gb, ea, eb, ta, tb):
    c = lax.axis_index("c")
    s = lax.axis_index("s")
    w = c * NS + s
    pltpu.sync_copy(pk_hbm.at[pl.ds(w * BPW, BPW)], pk_v)
    pltpu.sync_copy(deninv_hbm, dinv_v)
    pltpu.sync_copy(z2d_hbm, rows_b)
    nblk = NPAD // EBS // NS
    for b in range(nblk):
        pltpu.async_copy(rows_b, acc_sh.at[pl.ds((s * nblk + b) * EBS, EBS)],
                         tb)

    def unpack64(j, srcb, dstb):
        row = lax.shift_right_logical(j, 1)
        base = jnp.bitwise_and(j, 1) * EBS
        for k in range(EBS // L):
            sl = pl.ds(base + k * L, L)
            p16 = pk_v[row, sl]
            srcb[pl.ds(k * L, L)] = jnp.bitwise_and(p16, (1 << SHIFT) - 1)
            dstb[pl.ds(k * L, L)] = lax.shift_right_logical(p16, SHIFT)

    def exsrc(j):
        row = lax.shift_right_logical(w * NBS + j, 1)
        base = jnp.bitwise_and(j, 1) * EBS
        return exv_hbm.at[row, pl.ds(base, EBS)]

    unpack64(jnp.int32(0), sba, dba)
    pltpu.async_copy(hs_hbm.at[sba], rows_a, ga)
    pltpu.async_copy(exsrc(jnp.int32(0)), ex_a, ea)
    for b in range(nblk):
        pltpu.make_async_copy(
            rows_b, acc_sh.at[pl.ds((s * nblk + b) * EBS, EBS)], tb).wait()
    plsc.subcore_barrier()

    def half(j, drain_y, rows_x, sbx, dbx, ex_x, gx, ex_sx, tx,
             rows_y, sby, dby, ex_y, gy, ex_sy, ty):
        @pl.when(drain_y)
        def _():
            pltpu.make_async_copy(rows_y, acc_sh.at[dby], ty).wait()

        @pl.when(j + 1 < NBS)
        def _():
            unpack64(j + 1, sby, dby)
            pltpu.async_copy(hs_hbm.at[sby], rows_y, gy)
            pltpu.async_copy(exsrc(j + 1), ex_y, ex_sy)
        pltpu.make_async_copy(hs_hbm.at[sbx], rows_x, gx).wait()
        pltpu.make_async_copy(exsrc(j), ex_x, ex_sx).wait()
        for k in range(EBS // L):
            sl = pl.ds(k * L, L)
            d16 = dbx[sl]
            alpha_v[sl] = ex_x[sl] * plsc.load_gather(dinv_v, [d16])
        for r in range(EBS):
            ar = plsc.load_gather(alpha_v, [jnp.full((L,), r, jnp.int32)])
            for m in range(D // L):
                sl = pl.ds(m * L, L)
                rows_x[r, sl] = rows_x[r, sl] * ar
        pltpu.async_copy(rows_x, acc_sh.at[dbx], tx, add=True)

    def pair(g, carry):
        j = 2 * g
        half(j, g > 0, rows_a, sba, dba, ex_a, ga, ea, ta,
             rows_b, sbb, dbb, ex_b, gb, eb, tb)
        half(j + 1, g >= 0, rows_b, sbb, dbb, ex_b, gb, eb, tb,
             rows_a, sba, dba, ex_a, ga, ea, ta)
        return carry

    lax.fori_loop(0, NBS // 2, pair, 0)
    pltpu.make_async_copy(rows_b, acc_sh.at[dbb], tb).wait()
    plsc.subcore_barrier()
    rr = s * (NPAD // NS)
    pltpu.sync_copy(acc_sh.at[pl.ds(rr, NPAD // NS)],
                    out_hbm.at[c, pl.ds(rr, NPAD // NS)])


_sc_gat_agg = pl.kernel(
    _sc_gat_agg_body,
    compiler_params=_sc_params,
    out_type=jax.ShapeDtypeStruct((NC, NPAD, D), jnp.float32),
    mesh=_mesh,
    scratch_types=(
        [pltpu.VMEM((BPW, EB), jnp.int32)]
        + [pltpu.VMEM((EBS, D), jnp.float32)] * 2
        + [pltpu.VMEM((EBS,), jnp.int32)] * 4
        + [pltpu.VMEM((EBS,), jnp.float32)] * 2
        + [pltpu.VMEM((NPAD,), jnp.float32)]
        + [pltpu.VMEM((EBS,), jnp.float32)]
        + [pltpu.VMEM_SHARED((NPAD, D), jnp.float32)]
        + [pltpu.SemaphoreType.DMA] * 6
    ),
)


BT = 1024
NG = NPAD // BT


def _tc_dense_body(p_ref, hist_ref, xd_ref, wl_ref, bl_ref, wr_ref, o_ref):
    deg = jnp.sum(hist_ref[...], axis=0)
    dinv = 1.0 / jnp.maximum(deg, 1.0)
    agg = (p_ref[0] + p_ref[1]) * dinv[:, None]
    o_ref[...] = jax.nn.relu(
        jnp.dot(agg, wl_ref[...], preferred_element_type=jnp.float32)
        + bl_ref[...]
        + jnp.dot(xd_ref[...], wr_ref[...], preferred_element_type=jnp.float32))


def _tc_dense(p, hist, xd, wl, bl, wr):
    return pl.pallas_call(
        _tc_dense_body,
        grid=(NG,),
        in_specs=[
            pl.BlockSpec((NC, BT, D), lambda i: (0, i, 0)),
            pl.BlockSpec((NW, BT), lambda i: (0, i)),
            pl.BlockSpec((BT, D), lambda i: (i, 0)),
            pl.BlockSpec((D, D), lambda i: (0, 0)),
            pl.BlockSpec((1, D), lambda i: (0, 0)),
            pl.BlockSpec((D, D), lambda i: (0, 0)),
        ],
        out_specs=pl.BlockSpec((BT, D), lambda i: (i, 0)),
        out_shape=jax.ShapeDtypeStruct((NPAD, D), jnp.float32),
    )(p, hist, xd, wl, bl.reshape(1, D), wr)


def _tc_final_body(p_ref, hist_ref, ph_ref, bg_ref, wl_ref, bl_ref, wr_ref,
                   wm_ref, bm_ref, o_ref):
    deg = jnp.sum(hist_ref[...], axis=0)
    dinv = 1.0 / jnp.maximum(deg, 1.0)
    agg = (p_ref[0] + p_ref[1]) * dinv[:, None]
    h = jax.nn.relu(ph_ref[0] + ph_ref[1] + bg_ref[...])
    in_x = jax.nn.relu(
        jnp.dot(agg, wl_ref[...], preferred_element_type=jnp.float32)
        + bl_ref[...]
        + jnp.dot(h, wr_ref[...], preferred_element_type=jnp.float32))
    o_ref[...] = (jnp.dot(in_x, wm_ref[...], preferred_element_type=jnp.float32)
                  + bm_ref[...])


def _tc_final(p, hist, ph, bg, wl, bl, wr, wm, bm):
    return pl.pallas_call(
        _tc_final_body,
        grid=(NG,),
        in_specs=[
            pl.BlockSpec((NC, BT, D), lambda i: (0, i, 0)),
            pl.BlockSpec((NW, BT), lambda i: (0, i)),
            pl.BlockSpec((NC, BT, D), lambda i: (0, i, 0)),
            pl.BlockSpec((1, D), lambda i: (0, 0)),
            pl.BlockSpec((D, D), lambda i: (0, 0)),
            pl.BlockSpec((1, D), lambda i: (0, 0)),
            pl.BlockSpec((D, D), lambda i: (0, 0)),
            pl.BlockSpec((D, 1), lambda i: (0, 0)),
            pl.BlockSpec((1, 1), lambda i: (0, 0)),
        ],
        out_specs=pl.BlockSpec((BT, 1), lambda i: (i, 0)),
        out_shape=jax.ShapeDtypeStruct((NPAD, 1), jnp.float32),
    )(p, hist, ph, bg.reshape(1, D), wl, bl.reshape(1, D), wr,
      wm, bm.reshape(1, 1))


def _tc_attn_pre_body(g_ref, st_ref, ws_ref, wd_ref, as_ref, ad_ref,
                      hs_ref, hsa_ref, hda_ref, m_ref):
    i = pl.program_id(0)
    hs = jnp.dot(g_ref[...], ws_ref[...], preferred_element_type=jnp.float32)
    hs_ref[...] = hs
    hsa = jnp.dot(hs, as_ref[...], preferred_element_type=jnp.float32)
    wdv = jnp.dot(wd_ref[...], ad_ref[...], preferred_element_type=jnp.float32)
    hda = jnp.dot(st_ref[...], wdv, preferred_element_type=jnp.float32)
    hsa_ref[...] = hsa.reshape(BT // 128, 128)
    hda_ref[...] = hda.reshape(BT // 128, 128)

    @pl.when(i == 0)
    def _():
        m_ref[...] = jnp.full((1, 2), -1e30, jnp.float32)

    cur = jnp.concatenate(
        [jnp.max(hsa).reshape(1, 1), jnp.max(hda).reshape(1, 1)], axis=1)
    m_ref[...] = jnp.maximum(m_ref[...], cur)


def _tc_attn_pre(g, st, ws, wd, att_s, att_d):
    return pl.pallas_call(
        _tc_attn_pre_body,
        grid=(NG,),
        in_specs=[
            pl.BlockSpec((BT, D), lambda i: (i, 0)),
            pl.BlockSpec((BT, D), lambda i: (i, 0)),
            pl.BlockSpec((D, D), lambda i: (0, 0)),
            pl.BlockSpec((D, D), lambda i: (0, 0)),
            pl.BlockSpec((D, 1), lambda i: (0, 0)),
            pl.BlockSpec((D, 1), lambda i: (0, 0)),
        ],
        out_specs=(
            pl.BlockSpec((BT, D), lambda i: (i, 0)),
            pl.BlockSpec((BT // 128, 128), lambda i: (i, 0)),
            pl.BlockSpec((BT // 128, 128), lambda i: (i, 0)),
            pl.BlockSpec((1, 2), lambda i: (0, 0)),
        ),
        out_shape=(
            jax.ShapeDtypeStruct((NPAD, D), jnp.float32),
            jax.ShapeDtypeStruct((NPAD // 128, 128), jnp.float32),
            jax.ShapeDtypeStruct((NPAD // 128, 128), jnp.float32),
            jax.ShapeDtypeStruct((1, 2), jnp.float32),
        ),
    )(g, st, ws, wd, att_s.reshape(D, 1), att_d.reshape(D, 1))


EA_BT = 2048


def _tc_ea_body(ea_ref, wmat_ref, o_ref, m_ref):
    i = pl.program_id(0)
    o8 = jnp.dot(ea_ref[...], wmat_ref[...],
                 preferred_element_type=jnp.float32)
    o_ref[...] = o8

    @pl.when(i == 0)
    def _():
        m_ref[...] = jnp.full((1, 1), -1e30, jnp.float32)

    m_ref[...] = jnp.maximum(m_ref[...], jnp.max(o8).reshape(1, 1))


def _tc_ea(edge_attr, wg_e, att_e):
    we = jnp.dot(wg_e, att_e.reshape(D, 1),
                 preferred_element_type=jnp.float32).reshape(DE)
    wmat = (jnp.eye(8, dtype=jnp.float32)[:, None, :]
            * we[None, :, None]).reshape(128, 8)
    ea2 = jnp.concatenate(
        [edge_attr.reshape(E // 8, 128),
         jnp.zeros(((EPAD - E) // 8, 128), jnp.float32)], axis=0)
    o8, m3 = pl.pallas_call(
        _tc_ea_body,
        grid=(EPAD // 8 // EA_BT,),
        in_specs=[
            pl.BlockSpec((EA_BT, 128), lambda i: (i, 0)),
            pl.BlockSpec((128, 8), lambda i: (0, 0)),
        ],
        out_specs=(
            pl.BlockSpec((EA_BT, 8), lambda i: (i, 0)),
            pl.BlockSpec((1, 1), lambda i: (0, 0)),
        ),
        out_shape=(
            jax.ShapeDtypeStruct((EPAD // 8, 8), jnp.float32),
            jax.ShapeDtypeStruct((1, 1), jnp.float32),
        ),
    )(ea2, wmat)
    return o8.reshape(EROWS, EB), m3


def _tc_deninv_body(hist_ref, o_ref):
    den = jnp.sum(hist_ref[...], axis=0, keepdims=True)
    o_ref[...] = 1.0 / jnp.maximum(den, 1e-16)


def _tc_deninv(hist):
    return pl.pallas_call(
        _tc_deninv_body,
        out_shape=jax.ShapeDtypeStruct((1, NPAD), jnp.float32),
    )(hist)


def _pad_ei(ei):
    src = jnp.concatenate(
        [ei[0], jnp.zeros((EPAD - E,), ei.dtype)]).astype(jnp.int32)
    dst = jnp.concatenate(
        [ei[1], jnp.full((EPAD - E,), DUMP, ei.dtype)]).astype(jnp.int32)
    pk = jnp.bitwise_or(src, jnp.left_shift(dst, SHIFT))
    return (src.reshape(EROWS, EB), dst.reshape(EROWS, EB),
            pk.reshape(EROWS, EB))


def _pad_x(x):
    return jnp.concatenate([x, jnp.zeros((NPAD - N, D), x.dtype)], axis=0)


def kernel(x_game, x_state, edge_attr, Wl, bl, Wr, Wg_s, Wg_d, Wg_e,
           att_s, att_d, att_e, bg, Wm, bm, ei_gg, ei_ss, ei_hist, ei_in):
    z2d = jnp.zeros((EB, D), jnp.float32)
    z2ds = jnp.zeros((EBS, D), jnp.float32)
    z1d = jnp.zeros((NPAD,), jnp.float32)
    _, dgg, pgg = _pad_ei(ei_gg)
    _, dss, pss = _pad_ei(ei_ss)
    shh, dhh, phh = _pad_ei(ei_hist)
    _, din, pin_ = _pad_ei(ei_in)
    xg = _pad_x(x_game)
    xs = _pad_x(x_state)

    hist3 = _sc_hist(dgg, dss, din, z1d)
    hist_gg, hist_ss, hist_in = hist3[0], hist3[1], hist3[2]

    p = _sc_seg(xg, pgg, z2ds)
    g = _tc_dense(p, hist_gg, xg, Wl[0], bl[0], Wr[0])
    p = _sc_seg(g, pgg, z2ds)
    g = _tc_dense(p, hist_gg, g, Wl[1], bl[1], Wr[1])

    p = _sc_seg(xs, pss, z2ds)
    st = _tc_dense(p, hist_ss, xs, Wl[2], bl[2], Wr[2])
    p = _sc_seg(st, pss, z2ds)
    st = _tc_dense(p, hist_ss, st, Wl[3], bl[3], Wr[3])

    hs, hsa, hda, m12 = _tc_attn_pre(g, st, Wg_s, Wg_d, att_s, att_d)
    eav, m3 = _tc_ea(edge_attr, Wg_e, att_e)
    shift = jnp.maximum(m12[0, 0] + m12[0, 1] + m3[0, 0], 0.0)
    shift16 = jnp.broadcast_to(shift, (L,))
    ex, hist_den = _sc_gat_logits(shh, dhh, eav, hsa.reshape(NPAD),
                                  hda.reshape(NPAD), shift16, z1d)
    deninv = _tc_deninv(hist_den).reshape(NPAD)
    ph = _sc_gat_agg(hs, phh, ex, deninv, z2ds)

    p = _sc_seg(g, pin_, z2ds)
    out = _tc_final(p, hist_in, ph, bg, Wl[4], bl[4], Wr[4], Wm, bm)
    return out[:N]


# speedup vs baseline: 1.2375x; 1.0001x over previous
import functools

import jax
import jax.numpy as jnp
from jax import lax
from jax.experimental import pallas as pl
from jax.experimental.pallas import tpu as pltpu
from jax.experimental.pallas import tpu_sc as plsc

N = 10000
E = 320000
D = 128
DE = 16

NC = 2
NS = 16
NW = NC * NS
L = 16

NPAD = 10240
EB = 128
EW = 10240
BPW = EW // EB
EPAD = EW * NW
EROWS = EPAD // EB
DUMP = NPAD - 1
SHIFT = 14

_mesh = plsc.VectorSubcoreMesh(
    core_axis_name="c", subcore_axis_name="s", num_cores=NC, num_subcores=NS)
_sc_params = pltpu.CompilerParams(needs_layout_passes=False)


def _unpack(pk_v, j, srcb, dstb):
    for k in range(EB // L):
        sl = pl.ds(k * L, L)
        p16 = pk_v[j, sl]
        srcb[sl] = jnp.bitwise_and(p16, (1 << SHIFT) - 1)
        dstb[sl] = lax.shift_right_logical(p16, SHIFT)


EBS = 64
NBS = EW // EBS


def _sc_seg_body(x_hbm, pk_hbm, z2d_hbm,
                 out_hbm,
                 pk_v, r0, r1, r2, r3, s0, s1, s2, s3, d0, d1, d2, d3, acc_sh,
                 g0, g1, g2, g3, t0, t1, t2, t3):
    c = lax.axis_index("c")
    s = lax.axis_index("s")
    w = c * NS + s
    rows = (r0, r1, r2, r3)
    sb = (s0, s1, s2, s3)
    db = (d0, d1, d2, d3)
    gsem = (g0, g1, g2, g3)
    ssem = (t0, t1, t2, t3)
    pltpu.sync_copy(pk_hbm.at[pl.ds(w * BPW, BPW)], pk_v)
    pltpu.sync_copy(z2d_hbm, r3)
    nblk = NPAD // EBS // NS

    def unpack64(j, srcb, dstb):
        row = lax.shift_right_logical(j, 1)
        base = jnp.bitwise_and(j, 1) * EBS
        for k in range(EBS // L):
            sl = pl.ds(base + k * L, L)
            p16 = pk_v[row, sl]
            srcb[pl.ds(k * L, L)] = jnp.bitwise_and(p16, (1 << SHIFT) - 1)
            dstb[pl.ds(k * L, L)] = lax.shift_right_logical(p16, SHIFT)

    H2 = EBS // 2

    def gissue(q):
        pltpu.async_copy(x_hbm.at[sb[q].at[pl.ds(0, H2)]],
                         rows[q].at[pl.ds(0, H2)], gsem[q])
        pltpu.async_copy(x_hbm.at[sb[q].at[pl.ds(H2, H2)]],
                         rows[q].at[pl.ds(H2, H2)], gsem[q])

    def gwait(q):
        pltpu.make_async_copy(x_hbm.at[sb[q].at[pl.ds(0, H2)]],
                              rows[q].at[pl.ds(0, H2)], gsem[q]).wait()
        pltpu.make_async_copy(x_hbm.at[sb[q].at[pl.ds(H2, H2)]],
                              rows[q].at[pl.ds(H2, H2)], gsem[q]).wait()

    for q in range(3):
        unpack64(jnp.int32(q), sb[q], db[q])
        gissue(q)
    for b in range(nblk):
        pltpu.async_copy(r3, acc_sh.at[pl.ds((s * nblk + b) * EBS, EBS)], t3)
    for b in range(nblk):
        pltpu.make_async_copy(
            r3, acc_sh.at[pl.ds((s * nblk + b) * EBS, EBS)], t3).wait()
    plsc.subcore_barrier()

    def quad(g, carry):
        for q in range(4):
            j = 4 * g + q
            snx = (q + 3) % 4
            gwait(q)

            @pl.when(j > 0)
            def _():
                pltpu.make_async_copy(rows[snx], acc_sh.at[db[snx]],
                                      ssem[snx]).wait()

            pltpu.async_copy(rows[q], acc_sh.at[db[q]], ssem[q], add=True)

            @pl.when(j + 3 < NBS)
            def _():
                unpack64(j + 3, sb[snx], db[snx])
                gissue(snx)
        return carry

    lax.fori_loop(0, NBS // 4, quad, 0)
    pltpu.make_async_copy(rows[3], acc_sh.at[db[3]], ssem[3]).wait()
    plsc.subcore_barrier()
    rr = s * (NPAD // NS)
    pltpu.sync_copy(acc_sh.at[pl.ds(rr, NPAD // NS)],
                    out_hbm.at[c, pl.ds(rr, NPAD // NS)])


_sc_seg = pl.kernel(
    _sc_seg_body,
    compiler_params=_sc_params,
    out_type=jax.ShapeDtypeStruct((NC, NPAD, D), jnp.float32),
    mesh=_mesh,
    scratch_types=(
        [pltpu.VMEM((BPW, EB), jnp.int32)]
        + [pltpu.VMEM((EBS, D), jnp.float32)] * 4
        + [pltpu.VMEM((EBS,), jnp.int32)] * 8
        + [pltpu.VMEM_SHARED((NPAD, D), jnp.float32)]
        + [pltpu.SemaphoreType.DMA] * 8
    ),
)


def _sc_hist_body(d1_hbm, d2_hbm, d3_hbm, z1d_hbm,
                  hist_hbm,
                  dst_v, hist_v):
    c = lax.axis_index("c")
    s = lax.axis_index("s")
    w = c * NS + s
    ones = jnp.full((L,), 1.0, jnp.float32)
    for r, d_hbm in enumerate((d1_hbm, d2_hbm, d3_hbm)):
        pltpu.sync_copy(d_hbm.at[pl.ds(w * BPW, BPW)], dst_v)
        pltpu.sync_copy(z1d_hbm, hist_v)

        def body(j, carry):
            for k in range(EB // L):
                d16 = dst_v[j, pl.ds(k * L, L)]
                plsc.addupdate_scatter(hist_v, [d16], ones)
            return carry

        lax.fori_loop(0, BPW, body, 0)
        pltpu.sync_copy(hist_v, hist_hbm.at[r, w])


_sc_hist = pl.kernel(
    _sc_hist_body,
    compiler_params=_sc_params,
    out_type=jax.ShapeDtypeStruct((3, NW, NPAD), jnp.float32),
    mesh=_mesh,
    scratch_types=[
        pltpu.VMEM((BPW, EB), jnp.int32),
        pltpu.VMEM((NPAD,), jnp.float32),
    ],
)


def _sc_gat_logits_body(srcv_hbm, dstv_hbm, eav_hbm, hsa_hbm, hda_hbm,
                        shift_hbm, z1d_hbm,
                        ex_hbm, hist_hbm,
                        src_v, dst_v, ea_v, ex_v, hsa_v, hda_v, hist_v,
                        shift_v):
    c = lax.axis_index("c")
    s = lax.axis_index("s")
    w = c * NS + s
    pltpu.sync_copy(srcv_hbm.at[pl.ds(w * BPW, BPW)], src_v)
    pltpu.sync_copy(dstv_hbm.at[pl.ds(w * BPW, BPW)], dst_v)
    pltpu.sync_copy(eav_hbm.at[pl.ds(w * BPW, BPW)], ea_v)
    pltpu.sync_copy(hsa_hbm, hsa_v)
    pltpu.sync_copy(hda_hbm, hda_v)
    pltpu.sync_copy(z1d_hbm, hist_v)
    pltpu.sync_copy(shift_hbm, shift_v)

    def body(j, carry):
        for k in range(EB // L):
            sl = pl.ds(k * L, L)
            s16 = src_v[j, sl]
            d16 = dst_v[j, sl]
            a = (plsc.load_gather(hsa_v, [s16])
                 + plsc.load_gather(hda_v, [d16])
                 + ea_v[j, sl])
            a = jnp.maximum(a, 0.2 * a)
            ex = jnp.exp(a - shift_v[...])
            ex_v[j, sl] = ex
            plsc.addupdate_scatter(hist_v, [d16], ex)
        return carry

    lax.fori_loop(0, BPW, body, 0)
    pltpu.sync_copy(ex_v, ex_hbm.at[pl.ds(w * BPW, BPW)])
    pltpu.sync_copy(hist_v, hist_hbm.at[w])


_sc_gat_logits = pl.kernel(
    _sc_gat_logits_body,
    compiler_params=_sc_params,
    out_type=(
        jax.ShapeDtypeStruct((EROWS, EB), jnp.float32),
        jax.ShapeDtypeStruct((NW, NPAD), jnp.float32),
    ),
    mesh=_mesh,
    scratch_types=[
        pltpu.VMEM((BPW, EB), jnp.int32),
        pltpu.VMEM((BPW, EB), jnp.int32),
        pltpu.VMEM((BPW, EB), jnp.float32),
        pltpu.VMEM((BPW, EB), jnp.float32),
        pltpu.VMEM((NPAD,), jnp.float32),
        pltpu.VMEM((NPAD,), jnp.float32),
        pltpu.VMEM((NPAD,), jnp.float32),
        pltpu.VMEM((L,), jnp.float32),
    ],
)


def _sc_gat_agg_body(hs_hbm, pk_hbm, exv_hbm, deninv_hbm, z2d_hbm,
                     out_hbm,
                     pk_v, rows_a, rows_b, sba, sbb, dba, dbb, ex_a, ex_b,
                     dinv_v, alpha_v,
                     acc_sh, ga, gb, ea, eb, ta, tb):
    c = lax.axis_index("c")
    s = lax.axis_index("s")
    w = c * NS + s
    pltpu.sync_copy(pk_hbm.at[pl.ds(w * BPW, BPW)], pk_v)
    pltpu.sync_copy(deninv_hbm, dinv_v)
    pltpu.sync_copy(z2d_hbm, rows_b)
    nblk = NPAD // EBS // NS
    for b in range(nblk):
        pltpu.async_copy(rows_b, acc_sh.at[pl.ds((s * nblk + b) * EBS, EBS)],
                         tb)

    def unpack64(j, srcb, dstb):
        row = lax.shift_right_logical(j, 1)
        base = jnp.bitwise_and(j, 1) * EBS
        for k in range(EBS // L):
            sl = pl.ds(base + k * L, L)
            p16 = pk_v[row, sl]
            srcb[pl.ds(k * L, L)] = jnp.bitwise_and(p16, (1 << SHIFT) - 1)
            dstb[pl.ds(k * L, L)] = lax.shift_right_logical(p16, SHIFT)

    def exsrc(j):
        row = lax.shift_right_logical(w * NBS + j, 1)
        base = jnp.bitwise_and(j, 1) * EBS
        return exv_hbm.at[row, pl.ds(base, EBS)]

    unpack64(jnp.int32(0), sba, dba)
    pltpu.async_copy(hs_hbm.at[sba], rows_a, ga)
    pltpu.async_copy(exsrc(jnp.int32(0)), ex_a, ea)
    for b in range(nblk):
        pltpu.make_async_copy(
            rows_b, acc_sh.at[pl.ds((s * nblk + b) * EBS, EBS)], tb).wait()
    plsc.subcore_barrier()

    def half(j, drain_y, rows_x, sbx, dbx, ex_x, gx, ex_sx, tx,
             rows_y, sby, dby, ex_y, gy, ex_sy, ty):
        @pl.when(drain_y)
        def _():
            pltpu.make_async_copy(rows_y, acc_sh.at[dby], ty).wait()

        @pl.when(j + 1 < NBS)
        def _():
            unpack64(j + 1, sby, dby)
            pltpu.async_copy(hs_hbm.at[sby], rows_y, gy)
            pltpu.async_copy(exsrc(j + 1), ex_y, ex_sy)
        pltpu.make_async_copy(hs_hbm.at[sbx], rows_x, gx).wait()
        pltpu.make_async_copy(exsrc(j), ex_x, ex_sx).wait()
        for k in range(EBS // L):
            sl = pl.ds(k * L, L)
            d16 = dbx[sl]
            alpha_v[sl] = ex_x[sl] * plsc.load_gather(dinv_v, [d16])
        for r in range(EBS):
            ar = plsc.load_gather(alpha_v, [jnp.full((L,), r, jnp.int32)])
            for m in range(D // L):
                sl = pl.ds(m * L, L)
                rows_x[r, sl] = rows_x[r, sl] * ar
        pltpu.async_copy(rows_x, acc_sh.at[dbx], tx, add=True)

    def pair(g, carry):
        j = 2 * g
        half(j, g > 0, rows_a, sba, dba, ex_a, ga, ea, ta,
             rows_b, sbb, dbb, ex_b, gb, eb, tb)
        half(j + 1, g >= 0, rows_b, sbb, dbb, ex_b, gb, eb, tb,
             rows_a, sba, dba, ex_a, ga, ea, ta)
        return carry

    lax.fori_loop(0, NBS // 2, pair, 0)
    pltpu.make_async_copy(rows_b, acc_sh.at[dbb], tb).wait()
    plsc.subcore_barrier()
    rr = s * (NPAD // NS)
    pltpu.sync_copy(acc_sh.at[pl.ds(rr, NPAD // NS)],
                    out_hbm.at[c, pl.ds(rr, NPAD // NS)])


_sc_gat_agg = pl.kernel(
    _sc_gat_agg_body,
    compiler_params=_sc_params,
    out_type=jax.ShapeDtypeStruct((NC, NPAD, D), jnp.float32),
    mesh=_mesh,
    scratch_types=(
        [pltpu.VMEM((BPW, EB), jnp.int32)]
        + [pltpu.VMEM((EBS, D), jnp.float32)] * 2
        + [pltpu.VMEM((EBS,), jnp.int32)] * 4
        + [pltpu.VMEM((EBS,), jnp.float32)] * 2
        + [pltpu.VMEM((NPAD,), jnp.float32)]
        + [pltpu.VMEM((EBS,), jnp.float32)]
        + [pltpu.VMEM_SHARED((NPAD, D), jnp.float32)]
        + [pltpu.SemaphoreType.DMA] * 6
    ),
)


BT = 1024
NG = NPAD // BT


def _tc_dense_body(p_ref, hist_ref, xd_ref, wl_ref, bl_ref, wr_ref, o_ref):
    deg = jnp.sum(hist_ref[...], axis=0)
    dinv = 1.0 / jnp.maximum(deg, 1.0)
    agg = (p_ref[0] + p_ref[1]) * dinv[:, None]
    o_ref[...] = jax.nn.relu(
        jnp.dot(agg, wl_ref[...], preferred_element_type=jnp.float32)
        + bl_ref[...]
        + jnp.dot(xd_ref[...], wr_ref[...], preferred_element_type=jnp.float32))


def _tc_dense(p, hist, xd, wl, bl, wr):
    return pl.pallas_call(
        _tc_dense_body,
        grid=(NG,),
        in_specs=[
            pl.BlockSpec((NC, BT, D), lambda i: (0, i, 0)),
            pl.BlockSpec((NW, BT), lambda i: (0, i)),
            pl.BlockSpec((BT, D), lambda i: (i, 0)),
            pl.BlockSpec((D, D), lambda i: (0, 0)),
            pl.BlockSpec((1, D), lambda i: (0, 0)),
            pl.BlockSpec((D, D), lambda i: (0, 0)),
        ],
        out_specs=pl.BlockSpec((BT, D), lambda i: (i, 0)),
        out_shape=jax.ShapeDtypeStruct((NPAD, D), jnp.float32),
    )(p, hist, xd, wl, bl.reshape(1, D), wr)


def _tc_final_body(p_ref, hist_ref, ph_ref, bg_ref, wl_ref, bl_ref, wr_ref,
                   wm_ref, bm_ref, o_ref):
    deg = jnp.sum(hist_ref[...], axis=0)
    dinv = 1.0 / jnp.maximum(deg, 1.0)
    agg = (p_ref[0] + p_ref[1]) * dinv[:, None]
    h = jax.nn.relu(ph_ref[0] + ph_ref[1] + bg_ref[...])
    in_x = jax.nn.relu(
        jnp.dot(agg, wl_ref[...], preferred_element_type=jnp.float32)
        + bl_ref[...]
        + jnp.dot(h, wr_ref[...], preferred_element_type=jnp.float32))
    o_ref[...] = (jnp.dot(in_x, wm_ref[...], preferred_element_type=jnp.float32)
                  + bm_ref[...])


def _tc_final(p, hist, ph, bg, wl, bl, wr, wm, bm):
    return pl.pallas_call(
        _tc_final_body,
        grid=(NG,),
        in_specs=[
            pl.BlockSpec((NC, BT, D), lambda i: (0, i, 0)),
            pl.BlockSpec((NW, BT), lambda i: (0, i)),
            pl.BlockSpec((NC, BT, D), lambda i: (0, i, 0)),
            pl.BlockSpec((1, D), lambda i: (0, 0)),
            pl.BlockSpec((D, D), lambda i: (0, 0)),
            pl.BlockSpec((1, D), lambda i: (0, 0)),
            pl.BlockSpec((D, D), lambda i: (0, 0)),
            pl.BlockSpec((D, 1), lambda i: (0, 0)),
            pl.BlockSpec((1, 1), lambda i: (0, 0)),
        ],
        out_specs=pl.BlockSpec((BT, 1), lambda i: (i, 0)),
        out_shape=jax.ShapeDtypeStruct((NPAD, 1), jnp.float32),
    )(p, hist, ph, bg.reshape(1, D), wl, bl.reshape(1, D), wr,
      wm, bm.reshape(1, 1))


def _tc_attn_pre_body(g_ref, st_ref, ws_ref, wd_ref, as_ref, ad_ref,
                      hs_ref, hsa_ref, hda_ref, m_ref):
    i = pl.program_id(0)
    hs = jnp.dot(g_ref[...], ws_ref[...], preferred_element_type=jnp.float32)
    hs_ref[...] = hs
    hsa = jnp.dot(hs, as_ref[...], preferred_element_type=jnp.float32)
    wdv = jnp.dot(wd_ref[...], ad_ref[...], preferred_element_type=jnp.float32)
    hda = jnp.dot(st_ref[...], wdv, preferred_element_type=jnp.float32)
    hsa_ref[...] = hsa.reshape(BT // 128, 128)
    hda_ref[...] = hda.reshape(BT // 128, 128)

    @pl.when(i == 0)
    def _():
        m_ref[...] = jnp.full((1, 2), -1e30, jnp.float32)

    cur = jnp.concatenate(
        [jnp.max(hsa).reshape(1, 1), jnp.max(hda).reshape(1, 1)], axis=1)
    m_ref[...] = jnp.maximum(m_ref[...], cur)


def _tc_attn_pre(g, st, ws, wd, att_s, att_d):
    return pl.pallas_call(
        _tc_attn_pre_body,
        grid=(NG,),
        in_specs=[
            pl.BlockSpec((BT, D), lambda i: (i, 0)),
            pl.BlockSpec((BT, D), lambda i: (i, 0)),
            pl.BlockSpec((D, D), lambda i: (0, 0)),
            pl.BlockSpec((D, D), lambda i: (0, 0)),
            pl.BlockSpec((D, 1), lambda i: (0, 0)),
            pl.BlockSpec((D, 1), lambda i: (0, 0)),
        ],
        out_specs=(
            pl.BlockSpec((BT, D), lambda i: (i, 0)),
            pl.BlockSpec((BT // 128, 128), lambda i: (i, 0)),
            pl.BlockSpec((BT // 128, 128), lambda i: (i, 0)),
            pl.BlockSpec((1, 2), lambda i: (0, 0)),
        ),
        out_shape=(
            jax.ShapeDtypeStruct((NPAD, D), jnp.float32),
            jax.ShapeDtypeStruct((NPAD // 128, 128), jnp.float32),
            jax.ShapeDtypeStruct((NPAD // 128, 128), jnp.float32),
            jax.ShapeDtypeStruct((1, 2), jnp.float32),
        ),
    )(g, st, ws, wd, att_s.reshape(D, 1), att_d.reshape(D, 1))


EA_BT = 2048


def _tc_ea_body(ea_ref, wmat_ref, o_ref, m_ref):
    i = pl.program_id(0)
    o8 = jnp.dot(ea_ref[...], wmat_ref[...],
                 preferred_element_type=jnp.float32)
    o_ref[...] = o8

    @pl.when(i == 0)
    def _():
        m_ref[...] = jnp.full((1, 1), -1e30, jnp.float32)

    m_ref[...] = jnp.maximum(m_ref[...], jnp.max(o8).reshape(1, 1))


def _tc_ea(edge_attr, wg_e, att_e):
    we = jnp.dot(wg_e, att_e.reshape(D, 1),
                 preferred_element_type=jnp.float32).reshape(DE)
    wmat = (jnp.eye(8, dtype=jnp.float32)[:, None, :]
            * we[None, :, None]).reshape(128, 8)
    ea2 = jnp.concatenate(
        [edge_attr.reshape(E // 8, 128),
         jnp.zeros(((EPAD - E) // 8, 128), jnp.float32)], axis=0)
    o8, m3 = pl.pallas_call(
        _tc_ea_body,
        grid=(EPAD // 8 // EA_BT,),
        in_specs=[
            pl.BlockSpec((EA_BT, 128), lambda i: (i, 0)),
            pl.BlockSpec((128, 8), lambda i: (0, 0)),
        ],
        out_specs=(
            pl.BlockSpec((EA_BT, 8), lambda i: (i, 0)),
            pl.BlockSpec((1, 1), lambda i: (0, 0)),
        ),
        out_shape=(
            jax.ShapeDtypeStruct((EPAD // 8, 8), jnp.float32),
            jax.ShapeDtypeStruct((1, 1), jnp.float32),
        ),
    )(ea2, wmat)
    return o8.reshape(EROWS, EB), m3


def _tc_deninv_body(hist_ref, o_ref):
    den = jnp.sum(hist_ref[...], axis=0, keepdims=True)
    o_ref[...] = 1.0 / jnp.maximum(den, 1e-16)


def _tc_deninv(hist):
    return pl.pallas_call(
        _tc_deninv_body,
        out_shape=jax.ShapeDtypeStruct((1, NPAD), jnp.float32),
    )(hist)


def _pad_ei(ei):
    src = jnp.concatenate(
        [ei[0], jnp.zeros((EPAD - E,), ei.dtype)]).astype(jnp.int32)
    dst = jnp.concatenate(
        [ei[1], jnp.full((EPAD - E,), DUMP, ei.dtype)]).astype(jnp.int32)
    pk = jnp.bitwise_or(src, jnp.left_shift(dst, SHIFT))
    return (src.reshape(EROWS, EB), dst.reshape(EROWS, EB),
            pk.reshape(EROWS, EB))


def _pad_x(x):
    return jnp.concatenate([x, jnp.zeros((NPAD - N, D), x.dtype)], axis=0)


def kernel(x_game, x_state, edge_attr, Wl, bl, Wr, Wg_s, Wg_d, Wg_e,
           att_s, att_d, att_e, bg, Wm, bm, ei_gg, ei_ss, ei_hist, ei_in):
    z2d = jnp.zeros((EB, D), jnp.float32)
    z2ds = jnp.zeros((EBS, D), jnp.float32)
    z1d = jnp.zeros((NPAD,), jnp.float32)
    _, dgg, pgg = _pad_ei(ei_gg)
    _, dss, pss = _pad_ei(ei_ss)
    shh, dhh, phh = _pad_ei(ei_hist)
    _, din, pin_ = _pad_ei(ei_in)
    xg = _pad_x(x_game)
    xs = _pad_x(x_state)

    hist3 = _sc_hist(dgg, dss, din, z1d)
    hist_gg, hist_ss, hist_in = hist3[0], hist3[1], hist3[2]

    p = _sc_seg(xg, pgg, z2ds)
    g = _tc_dense(p, hist_gg, xg, Wl[0], bl[0], Wr[0])
    p = _sc_seg(g, pgg, z2ds)
    g = _tc_dense(p, hist_gg, g, Wl[1], bl[1], Wr[1])

    p = _sc_seg(xs, pss, z2ds)
    st = _tc_dense(p, hist_ss, xs, Wl[2], bl[2], Wr[2])
    p = _sc_seg(st, pss, z2ds)
    st = _tc_dense(p, hist_ss, st, Wl[3], bl[3], Wr[3])

    hs, hsa, hda, m12 = _tc_attn_pre(g, st, Wg_s, Wg_d, att_s, att_d)
    eav, m3 = _tc_ea(edge_attr, Wg_e, att_e)
    shift = jnp.maximum(m12[0, 0] + m12[0, 1] + m3[0, 0], 0.0)
    shift16 = jnp.broadcast_to(shift, (L,))
    ex, hist_den = _sc_gat_logits(shh, dhh, eav, hsa.reshape(NPAD),
                                  hda.reshape(NPAD), shift16, z1d)
    deninv = _tc_deninv(hist_den).reshape(NPAD)
    ph = _sc_gat_agg(hs, phh, ex, deninv, z2ds)

    p = _sc_seg(g, pin_, z2ds)
    out = _tc_final(p, hist_in, ph, bg, Wl[4], bl[4], Wr[4], Wm, bm)
    return out[:N]
